# Initial kernel scaffold; baseline (speedup 1.0000x reference)
import jax
import jax.numpy as jnp
from jax.experimental import pallas as pl


def kernel(sender_features, receiver_features, edge_features, senders, receivers, conditioning, msg_W1, msg_b1, msg_W2, msg_b2, msg_W3, msg_b3, upd_W1, upd_b1, upd_W2, upd_b2, upd_W3, upd_b3, ln_gamma, ln_beta):
    raise NotImplementedError("write your pallas kernel here")


# speedup vs baseline: 15.3007x; 15.3007x over previous
import functools

import jax
import jax.numpy as jnp
from jax import lax
from jax.experimental import pallas as pl
from jax.experimental.pallas import tpu as pltpu
from jax.experimental.pallas import tpu_sc as plsc

NUM_TILES = 16
CH = 80


def _proj_body(s_ref, r_ref, ws_ref, wr_ref, sp_ref, rp_ref):
    sp_ref[0] = jnp.dot(s_ref[0], ws_ref[...], preferred_element_type=jnp.float32)
    rp_ref[0] = jnp.dot(r_ref[0], wr_ref[...], preferred_element_type=jnp.float32)


def _node_projections(sender_features, receiver_features, w1s, w1r, bn):
    B, NS, SD = sender_features.shape
    H = w1s.shape[1]
    grid = (B, NS // bn)
    return pl.pallas_call(
        _proj_body,
        grid=grid,
        in_specs=[
            pl.BlockSpec((1, bn, SD), lambda b, j: (b, j, 0)),
            pl.BlockSpec((1, bn, SD), lambda b, j: (b, j, 0)),
            pl.BlockSpec((SD, H), lambda b, j: (0, 0)),
            pl.BlockSpec((SD, H), lambda b, j: (0, 0)),
        ],
        out_specs=[
            pl.BlockSpec((1, bn, H), lambda b, j: (b, j, 0)),
            pl.BlockSpec((1, bn, H), lambda b, j: (b, j, 0)),
        ],
        out_shape=[
            jax.ShapeDtypeStruct((B, NS, H), jnp.float32),
            jax.ShapeDtypeStruct((B, NS, H), jnp.float32),
        ],
    )(sender_features, receiver_features, w1s, w1r)


SUP = 25


def _make_gather_kernel(B, E, NS, NR, H):
    nch = E // (NUM_TILES * CH)
    nsup = nch // SUP
    nzc = NR // CH
    zc_per_tile = -(-nzc // NUM_TILES)
    mesh = plsc.VectorSubcoreMesh(core_axis_name="c", subcore_axis_name="s")

    @functools.partial(
        pl.kernel,
        mesh=mesh,
        out_type=(
            jax.ShapeDtypeStruct((B * E, H), jnp.float32),
            jax.ShapeDtypeStruct((NR, H), jnp.float32),
        ),
        scratch_types=[
            pltpu.VMEM((SUP, 1, CH), jnp.int32),
            pltpu.VMEM((SUP, 1, CH), jnp.int32),
            pltpu.VMEM((CH, H), jnp.float32),
            pltpu.VMEM((CH, H), jnp.float32),
            pltpu.VMEM((CH, H), jnp.float32),
            pltpu.VMEM_SHARED((NR, H), jnp.float32),
            pltpu.SemaphoreType.DMA,
            pltpu.SemaphoreType.DMA,
        ],
    )
    def gather_kernel(sp0, rp0, sp1, rp1, s4d, r4d, g_hbm, cnt_hbm,
                      sidx, ridx, srows, rrows, obuf, cacc, sem_s, sem_r):
        c = lax.axis_index("c")
        s = lax.axis_index("s")

        def zrow(i, _):
            for k in range(H // 16):
                obuf[i, pl.ds(k * 16, 16)] = jnp.zeros((16,), jnp.float32)
            return None
        lax.fori_loop(0, CH, zrow, None)

        def zchunk(t, _):
            j = s + t * NUM_TILES

            @pl.when((j < nzc) & (c == 0))
            def _():
                pltpu.sync_copy(obuf, cacc.at[pl.ds(j * CH, CH)])
            return None
        lax.fori_loop(0, zc_per_tile, zchunk, None)

        def orow(i, _):
            obuf[i, pl.ds(0, 16)] = jnp.ones((16,), jnp.float32)
            return None
        lax.fori_loop(0, CH, orow, None)
        plsc.subcore_barrier()

        def run(sp, rp):
            def sup_chunk(u, _):
                pltpu.sync_copy(s4d.at[s, pl.ds(u * SUP, SUP)], sidx)
                pltpu.sync_copy(r4d.at[s, pl.ds(u * SUP, SUP)], ridx)

                def chunk(jj, _):
                    cp_s = pltpu.async_copy(sp.at[sidx.at[jj, 0]], srows, sem_s)
                    cp_r = pltpu.async_copy(rp.at[ridx.at[jj, 0]], rrows, sem_r)
                    cp_s.wait()
                    cp_r.wait()

                    def add_row(i, _):
                        for k in range(H // 16):
                            sl = pl.ds(k * 16, 16)
                            srows[i, sl] = srows[i, sl] + rrows[i, sl]
                        return None

                    lax.fori_loop(0, CH, add_row, None, unroll=2)
                    e0 = c * E + s * (nch * CH) + (u * SUP + jj) * CH
                    pltpu.sync_copy(srows, g_hbm.at[pl.ds(e0, CH)])

                    @pl.when(c == 0)
                    def _():
                        pltpu.sync_copy(obuf, cacc.at[ridx.at[jj, 0]], add=True)
                    return None

                lax.fori_loop(0, SUP, chunk, None)
                return None

            lax.fori_loop(0, nsup, sup_chunk, None)

        @pl.when(c == 0)
        def _():
            run(sp0, rp0)

        @pl.when(c == 1)
        def _():
            run(sp1, rp1)

        plsc.subcore_barrier()

        def wchunk(t, _):
            j = s + t * NUM_TILES

            @pl.when((j < nzc) & (c == 0))
            def _():
                pltpu.sync_copy(cacc.at[pl.ds(j * CH, CH)], srows)
                pltpu.sync_copy(srows, cnt_hbm.at[pl.ds(j * CH, CH)])
            return None
        lax.fori_loop(0, zc_per_tile, wchunk, None)

    return gather_kernel


def _edge_mlp_body(g_ref, ef_ref, cond_ref, w1e_ref, w1c_ref, b1_ref,
                   w2_ref, b2_ref, w3_ref, b3_ref, msg_ref):
    ep = jnp.dot(ef_ref[...], w1e_ref[...], preferred_element_type=jnp.float32)
    cond_row = cond_ref[pl.ds(pl.program_id(0), 1), :]
    cp = jnp.dot(cond_row, w1c_ref[...], preferred_element_type=jnp.float32)
    h1 = jnp.maximum(g_ref[0] + ep + cp + b1_ref[...], 0.0)
    h2 = jnp.maximum(
        jnp.dot(h1, w2_ref[...], preferred_element_type=jnp.float32) + b2_ref[...], 0.0)
    msg_ref[0] = jnp.dot(h2, w3_ref[...], preferred_element_type=jnp.float32) + b3_ref[...]


def _edge_mlp(g, edge_features, conditioning, w1e, w1c, b1, w2, b2, w3, b3, be):
    B, E, H = g.shape
    ED = edge_features.shape[1]
    CD = conditioning.shape[1]
    grid = (B, E // be)
    return pl.pallas_call(
        _edge_mlp_body,
        grid=grid,
        in_specs=[
            pl.BlockSpec((1, be, H), lambda b, j: (b, j, 0)),
            pl.BlockSpec((be, ED), lambda b, j: (j, 0)),
            pl.BlockSpec((B, CD), lambda b, j: (0, 0)),
            pl.BlockSpec((ED, H), lambda b, j: (0, 0)),
            pl.BlockSpec((CD, H), lambda b, j: (0, 0)),
            pl.BlockSpec((1, H), lambda b, j: (0, 0)),
            pl.BlockSpec((H, H), lambda b, j: (0, 0)),
            pl.BlockSpec((1, H), lambda b, j: (0, 0)),
            pl.BlockSpec((H, H), lambda b, j: (0, 0)),
            pl.BlockSpec((1, H), lambda b, j: (0, 0)),
        ],
        out_specs=pl.BlockSpec((1, be, H), lambda b, j: (b, j, 0)),
        out_shape=jax.ShapeDtypeStruct((B, E, H), jnp.float32),
    )(g, edge_features, conditioning, w1e, w1c, b1, w2, b2, w3, b3)


def _make_scatter_kernel(B, E, NR, H):
    nch = E // (NUM_TILES * CH)
    nzc = NR // CH
    zc_per_tile = -(-nzc // NUM_TILES)
    mesh = plsc.VectorSubcoreMesh(core_axis_name="c", subcore_axis_name="s")

    @functools.partial(
        pl.kernel,
        mesh=mesh,
        out_type=jax.ShapeDtypeStruct((B * NR, H), jnp.float32),
        scratch_types=[
            pltpu.VMEM((nch, CH), jnp.int32),
            pltpu.VMEM((CH, H), jnp.float32),
            pltpu.VMEM_SHARED((NR, H), jnp.float32),
        ],
    )
    def scatter_kernel(msg_hbm, r3d, agg_hbm, ridx, mrows, acc):
        c = lax.axis_index("c")
        s = lax.axis_index("s")

        def zrow(i, _):
            for k in range(H // 16):
                mrows[i, pl.ds(k * 16, 16)] = jnp.zeros((16,), jnp.float32)
            return None
        lax.fori_loop(0, CH, zrow, None)

        def zchunk(t, _):
            j = s + t * NUM_TILES

            @pl.when(j < nzc)
            def _():
                pltpu.sync_copy(mrows, acc.at[pl.ds(j * CH, CH)])
            return None
        lax.fori_loop(0, zc_per_tile, zchunk, None)
        plsc.subcore_barrier()

        pltpu.sync_copy(r3d.at[s], ridx)

        def chunk(j, _):
            e0 = c * E + s * (nch * CH) + j * CH
            pltpu.sync_copy(msg_hbm.at[pl.ds(e0, CH)], mrows)
            pltpu.sync_copy(mrows, acc.at[ridx.at[j]], add=True)
            return None
        lax.fori_loop(0, nch, chunk, None)
        plsc.subcore_barrier()

        def wchunk(t, _):
            j = s + t * NUM_TILES

            @pl.when(j < nzc)
            def _():
                pltpu.sync_copy(acc.at[pl.ds(j * CH, CH)], mrows)
                pltpu.sync_copy(mrows, agg_hbm.at[pl.ds(c * NR + j * CH, CH)])
            return None
        lax.fori_loop(0, zc_per_tile, wchunk, None)

    return scatter_kernel


def _node_update_body(rf_ref, agg_ref, cnt_ref, cond_ref, lng_ref, lnb_ref,
                      u1_ref, c1_ref, u2_ref, c2_ref, u3_ref, c3_ref, out_ref,
                      *, RD, H, CD):
    upd_in = RD + H + CD
    rf = rf_ref[0]
    counts = jnp.maximum(cnt_ref[...][:, 0:1], 1.0)
    a = agg_ref[0] / counts
    cond = cond_ref[pl.ds(pl.program_id(0), 1), :]

    mu = (jnp.sum(rf, axis=1, keepdims=True) + jnp.sum(a, axis=1, keepdims=True)
          + jnp.sum(cond)) / upd_in
    m2 = (jnp.sum(rf * rf, axis=1, keepdims=True)
          + jnp.sum(a * a, axis=1, keepdims=True) + jnp.sum(cond * cond)) / upd_in
    var = m2 - mu * mu
    rstd = lax.rsqrt(var + 1e-5)

    nr = (rf - mu) * rstd * lng_ref[:, 0:RD] + lnb_ref[:, 0:RD]
    na = (a - mu) * rstd * lng_ref[:, RD:RD + H] + lnb_ref[:, RD:RD + H]
    nc = (cond - mu) * rstd * lng_ref[:, RD + H:] + lnb_ref[:, RD + H:]

    h = jnp.dot(nr, u1_ref[0:RD, :], preferred_element_type=jnp.float32)
    h += jnp.dot(na, u1_ref[RD:RD + H, :], preferred_element_type=jnp.float32)
    h += jnp.dot(nc, u1_ref[RD + H:, :], preferred_element_type=jnp.float32)
    h = jnp.maximum(h + c1_ref[...], 0.0)
    h = jnp.maximum(
        jnp.dot(h, u2_ref[...], preferred_element_type=jnp.float32) + c2_ref[...], 0.0)
    out_ref[0] = rf + jnp.dot(h, u3_ref[...], preferred_element_type=jnp.float32) + c3_ref[...]


def _node_update(receiver_features, aggsum, cnt_t, conditioning, lng, lnb,
                 u1, c1, u2, c2, u3, c3, bn):
    B, NR, RD = receiver_features.shape
    H = aggsum.shape[2]
    CD = conditioning.shape[1]
    upd_in = RD + H + CD
    grid = (B, NR // bn)
    body = functools.partial(_node_update_body, RD=RD, H=H, CD=CD)
    return pl.pallas_call(
        body,
        grid=grid,
        in_specs=[
            pl.BlockSpec((1, bn, RD), lambda b, j: (b, j, 0)),
            pl.BlockSpec((1, bn, H), lambda b, j: (b, j, 0)),
            pl.BlockSpec((bn, H), lambda b, j: (j, 0)),
            pl.BlockSpec((B, CD), lambda b, j: (0, 0)),
            pl.BlockSpec((1, upd_in), lambda b, j: (0, 0)),
            pl.BlockSpec((1, upd_in), lambda b, j: (0, 0)),
            pl.BlockSpec((upd_in, H), lambda b, j: (0, 0)),
            pl.BlockSpec((1, H), lambda b, j: (0, 0)),
            pl.BlockSpec((H, H), lambda b, j: (0, 0)),
            pl.BlockSpec((1, H), lambda b, j: (0, 0)),
            pl.BlockSpec((H, RD), lambda b, j: (0, 0)),
            pl.BlockSpec((1, RD), lambda b, j: (0, 0)),
        ],
        out_specs=pl.BlockSpec((1, bn, RD), lambda b, j: (b, j, 0)),
        out_shape=jax.ShapeDtypeStruct((B, NR, RD), jnp.float32),
    )(receiver_features, aggsum, cnt_t, conditioning, lng, lnb,
      u1, c1, u2, c2, u3, c3)


def kernel(sender_features, receiver_features, edge_features, senders, receivers,
           conditioning, msg_W1, msg_b1, msg_W2, msg_b2, msg_W3, msg_b3,
           upd_W1, upd_b1, upd_W2, upd_b2, upd_W3, upd_b3, ln_gamma, ln_beta):
    B, NS, SD = sender_features.shape
    _, NR, RD = receiver_features.shape
    E, ED = edge_features.shape
    CD = conditioning.shape[1]
    H = msg_W2.shape[0]
    assert B == 2 and NS == NR and SD == RD
    assert E % (NUM_TILES * CH) == 0 and NR % CH == 0
    assert E % (NUM_TILES * 16) == 0 and NR % 16 == 0

    w1s = msg_W1[:SD]
    w1r = msg_W1[SD:SD + RD]
    w1e = msg_W1[SD + RD:SD + RD + ED]
    w1c = msg_W1[SD + RD + ED:]
    b1 = msg_b1.reshape(1, H)
    b2 = msg_b2.reshape(1, H)
    b3 = msg_b3.reshape(1, H)
    c1 = upd_b1.reshape(1, H)
    c2 = upd_b2.reshape(1, H)
    c3 = upd_b3.reshape(1, RD)
    lng = ln_gamma.reshape(1, -1)
    lnb = ln_beta.reshape(1, -1)
    nch = E // (NUM_TILES * CH)
    s4d = senders.astype(jnp.int32).reshape(NUM_TILES, nch, 1, CH)
    r4d = receivers.astype(jnp.int32).reshape(NUM_TILES, nch, 1, CH)
    r3d = receivers.astype(jnp.int32).reshape(NUM_TILES, nch, CH)

    sp, rp = _node_projections(sender_features, receiver_features, w1s, w1r, bn=2000)

    gather = _make_gather_kernel(B, E, NS, NR, H)
    g, cnt = gather(sp[0], rp[0], sp[1], rp[1], s4d, r4d)
    g = g.reshape(B, E, H)

    msg = _edge_mlp(g, edge_features, conditioning, w1e, w1c, b1,
                    msg_W2, b2, msg_W3, b3, be=2000)

    scatter = _make_scatter_kernel(B, E, NR, H)
    aggsum = scatter(msg.reshape(B * E, H), r3d).reshape(B, NR, H)

    return _node_update(receiver_features, aggsum, cnt, conditioning, lng, lnb,
                        upd_W1, c1, upd_W2, c2, upd_W3, c3, bn=2000)


# speedup vs baseline: 16.7549x; 1.0950x over previous
import functools

import jax
import jax.numpy as jnp
from jax import lax
from jax.experimental import pallas as pl
from jax.experimental.pallas import tpu as pltpu
from jax.experimental.pallas import tpu_sc as plsc

NUM_TILES = 16
CHS = 80
CHG = 40
SUPG = 50


def _proj_body(s_ref, r_ref, ws_ref, wr_ref, sp_ref, rp_ref):
    sp_ref[0] = jnp.dot(s_ref[0], ws_ref[...], preferred_element_type=jnp.float32)
    rp_ref[0] = jnp.dot(r_ref[0], wr_ref[...], preferred_element_type=jnp.float32)


def _node_projections(sender_features, receiver_features, w1s, w1r, bn):
    B, NS, SD = sender_features.shape
    H = w1s.shape[1]
    grid = (B, NS // bn)
    return pl.pallas_call(
        _proj_body,
        grid=grid,
        in_specs=[
            pl.BlockSpec((1, bn, SD), lambda b, j: (b, j, 0)),
            pl.BlockSpec((1, bn, SD), lambda b, j: (b, j, 0)),
            pl.BlockSpec((SD, H), lambda b, j: (0, 0)),
            pl.BlockSpec((SD, H), lambda b, j: (0, 0)),
        ],
        out_specs=[
            pl.BlockSpec((1, bn, H), lambda b, j: (b, j, 0)),
            pl.BlockSpec((1, bn, H), lambda b, j: (b, j, 0)),
        ],
        out_shape=[
            jax.ShapeDtypeStruct((B, NS, H), jnp.float32),
            jax.ShapeDtypeStruct((B, NS, H), jnp.float32),
        ],
    )(sender_features, receiver_features, w1s, w1r)


def _make_gather_kernel(B, E, NS, NR, H):
    nch = E // (NUM_TILES * CHG)
    nsup = nch // SUPG
    npair = SUPG // 2
    nzc = NR // CHG
    zc_per_tile = -(-nzc // NUM_TILES)
    per_tile = nch * CHG
    mesh = plsc.VectorSubcoreMesh(core_axis_name="c", subcore_axis_name="s")

    @functools.partial(
        pl.kernel,
        mesh=mesh,
        out_type=(
            jax.ShapeDtypeStruct((B * E, H), jnp.float32),
            jax.ShapeDtypeStruct((2 * NR, H), jnp.float32),
        ),
        scratch_types=[
            pltpu.VMEM((SUPG, 1, CHG), jnp.int32),
            pltpu.VMEM((SUPG, 1, CHG), jnp.int32),
            pltpu.VMEM((CHG, H), jnp.float32),
            pltpu.VMEM((CHG, H), jnp.float32),
            pltpu.VMEM((CHG, H), jnp.float32),
            pltpu.VMEM((CHG, H), jnp.float32),
            pltpu.VMEM((CHG, H), jnp.float32),
            pltpu.VMEM_SHARED((NR, H), jnp.float32),
            pltpu.SemaphoreType.DMA,
            pltpu.SemaphoreType.DMA,
            pltpu.SemaphoreType.DMA,
            pltpu.SemaphoreType.DMA,
        ],
    )
    def gather_kernel(sp0, rp0, sp1, rp1, s4d, r4d, g_hbm, cnt_hbm,
                      sidx, ridx, sa, ra, sb, rb, obuf, cacc,
                      sem_a, sem_b, sem_wa, sem_wb):
        c = lax.axis_index("c")
        s = lax.axis_index("s")

        def zrow(i, _):
            for k in range(H // 16):
                obuf[i, pl.ds(k * 16, 16)] = jnp.zeros((16,), jnp.float32)
            return None
        lax.fori_loop(0, CHG, zrow, None)

        def zchunk(t, _):
            j = s + t * NUM_TILES

            @pl.when(j < nzc)
            def _():
                pltpu.sync_copy(obuf, cacc.at[pl.ds(j * CHG, CHG)])
            return None
        lax.fori_loop(0, zc_per_tile, zchunk, None)

        def orow(i, _):
            obuf[i, pl.ds(0, 16)] = jnp.ones((16,), jnp.float32)
            return None
        lax.fori_loop(0, CHG, orow, None)
        plsc.subcore_barrier()

        def run(sp, rp):
            def gissue(jj, sbuf, rbuf, sem):
                pltpu.async_copy(sp.at[sidx.at[jj, 0]], sbuf, sem)
                pltpu.async_copy(rp.at[ridx.at[jj, 0]], rbuf, sem)

            def gwait(sbuf, rbuf, sem):
                pltpu.make_async_copy(sp.at[sidx.at[0, 0]], sbuf, sem).wait()
                pltpu.make_async_copy(rp.at[ridx.at[0, 0]], rbuf, sem).wait()

            def add(sbuf, rbuf):
                def add_row(i, _):
                    for k in range(H // 16):
                        sl = pl.ds(k * 16, 16)
                        sbuf[i, sl] = sbuf[i, sl] + rbuf[i, sl]
                    return None
                lax.fori_loop(0, CHG, add_row, None, unroll=2)

            def wstart(jabs, sbuf, sem):
                e0 = c * E + s * per_tile + jabs * CHG
                pltpu.async_copy(sbuf, g_hbm.at[pl.ds(e0, CHG)], sem)

            def wwait(sbuf, sem):
                pltpu.make_async_copy(sbuf, g_hbm.at[pl.ds(0, CHG)], sem).wait()

            def sup(u, _):
                pltpu.sync_copy(s4d.at[s, pl.ds(u * SUPG, SUPG)], sidx)
                pltpu.sync_copy(r4d.at[s, pl.ds(u * SUPG, SUPG)], ridx)
                gissue(0, sa, ra, sem_a)
                gissue(1, sb, rb, sem_b)
                base = u * SUPG

                def pair(p, _):
                    gwait(sa, ra, sem_a)
                    add(sa, ra)
                    wstart(base + 2 * p, sa, sem_wa)
                    gwait(sb, rb, sem_b)
                    add(sb, rb)
                    wstart(base + 2 * p + 1, sb, sem_wb)
                    wwait(sa, sem_wa)

                    @pl.when(p < npair - 1)
                    def _():
                        gissue(2 * p + 2, sa, ra, sem_a)
                    wwait(sb, sem_wb)

                    @pl.when(p < npair - 1)
                    def _():
                        gissue(2 * p + 3, sb, rb, sem_b)

                    pltpu.sync_copy(obuf, cacc.at[ridx.at[2 * p + c, 0]], add=True)
                    return None

                lax.fori_loop(0, npair, pair, None)
                return None

            lax.fori_loop(0, nsup, sup, None)

        @pl.when(c == 0)
        def _():
            run(sp0, rp0)

        @pl.when(c == 1)
        def _():
            run(sp1, rp1)

        plsc.subcore_barrier()

        def wchunk(t, _):
            j = s + t * NUM_TILES

            @pl.when(j < nzc)
            def _():
                pltpu.sync_copy(cacc.at[pl.ds(j * CHG, CHG)], sa)
                pltpu.sync_copy(sa, cnt_hbm.at[pl.ds(c * NR + j * CHG, CHG)])
            return None
        lax.fori_loop(0, zc_per_tile, wchunk, None)

    return gather_kernel


def _edge_mlp_body(g_ref, ef_ref, cond_ref, w1e_ref, w1c_ref, b1_ref,
                   w2_ref, b2_ref, w3_ref, b3_ref, msg_ref):
    ep = jnp.dot(ef_ref[...], w1e_ref[...], preferred_element_type=jnp.float32)
    cond_row = cond_ref[pl.ds(pl.program_id(0), 1), :]
    cp = jnp.dot(cond_row, w1c_ref[...], preferred_element_type=jnp.float32)
    h1 = jnp.maximum(g_ref[0] + ep + cp + b1_ref[...], 0.0)
    h2 = jnp.maximum(
        jnp.dot(h1, w2_ref[...], preferred_element_type=jnp.float32) + b2_ref[...], 0.0)
    msg_ref[0] = jnp.dot(h2, w3_ref[...], preferred_element_type=jnp.float32) + b3_ref[...]


def _edge_mlp(g, edge_features, conditioning, w1e, w1c, b1, w2, b2, w3, b3, be):
    B, E, H = g.shape
    ED = edge_features.shape[1]
    CD = conditioning.shape[1]
    grid = (B, E // be)
    return pl.pallas_call(
        _edge_mlp_body,
        grid=grid,
        in_specs=[
            pl.BlockSpec((1, be, H), lambda b, j: (b, j, 0)),
            pl.BlockSpec((be, ED), lambda b, j: (j, 0)),
            pl.BlockSpec((B, CD), lambda b, j: (0, 0)),
            pl.BlockSpec((ED, H), lambda b, j: (0, 0)),
            pl.BlockSpec((CD, H), lambda b, j: (0, 0)),
            pl.BlockSpec((1, H), lambda b, j: (0, 0)),
            pl.BlockSpec((H, H), lambda b, j: (0, 0)),
            pl.BlockSpec((1, H), lambda b, j: (0, 0)),
            pl.BlockSpec((H, H), lambda b, j: (0, 0)),
            pl.BlockSpec((1, H), lambda b, j: (0, 0)),
        ],
        out_specs=pl.BlockSpec((1, be, H), lambda b, j: (b, j, 0)),
        out_shape=jax.ShapeDtypeStruct((B, E, H), jnp.float32),
    )(g, edge_features, conditioning, w1e, w1c, b1, w2, b2, w3, b3)


def _make_scatter_kernel(B, E, NR, H):
    nch = E // (NUM_TILES * CHS)
    nzc = NR // CHS
    zc_per_tile = -(-nzc // NUM_TILES)
    mesh = plsc.VectorSubcoreMesh(core_axis_name="c", subcore_axis_name="s")

    @functools.partial(
        pl.kernel,
        mesh=mesh,
        out_type=jax.ShapeDtypeStruct((B * NR, H), jnp.float32),
        scratch_types=[
            pltpu.VMEM((nch, CHS), jnp.int32),
            pltpu.VMEM((CHS, H), jnp.float32),
            pltpu.VMEM_SHARED((NR, H), jnp.float32),
        ],
    )
    def scatter_kernel(msg_hbm, r3d, agg_hbm, ridx, mrows, acc):
        c = lax.axis_index("c")
        s = lax.axis_index("s")

        def zrow(i, _):
            for k in range(H // 16):
                mrows[i, pl.ds(k * 16, 16)] = jnp.zeros((16,), jnp.float32)
            return None
        lax.fori_loop(0, CHS, zrow, None)

        def zchunk(t, _):
            j = s + t * NUM_TILES

            @pl.when(j < nzc)
            def _():
                pltpu.sync_copy(mrows, acc.at[pl.ds(j * CHS, CHS)])
            return None
        lax.fori_loop(0, zc_per_tile, zchunk, None)
        plsc.subcore_barrier()

        pltpu.sync_copy(r3d.at[s], ridx)

        def chunk(j, _):
            e0 = c * E + s * (nch * CHS) + j * CHS
            pltpu.sync_copy(msg_hbm.at[pl.ds(e0, CHS)], mrows)
            pltpu.sync_copy(mrows, acc.at[ridx.at[j]], add=True)
            return None
        lax.fori_loop(0, nch, chunk, None)
        plsc.subcore_barrier()

        def wchunk(t, _):
            j = s + t * NUM_TILES

            @pl.when(j < nzc)
            def _():
                pltpu.sync_copy(acc.at[pl.ds(j * CHS, CHS)], mrows)
                pltpu.sync_copy(mrows, agg_hbm.at[pl.ds(c * NR + j * CHS, CHS)])
            return None
        lax.fori_loop(0, zc_per_tile, wchunk, None)

    return scatter_kernel


def _node_update_body(rf_ref, agg_ref, cnt_ref, cond_ref, lng_ref, lnb_ref,
                      u1_ref, c1_ref, u2_ref, c2_ref, u3_ref, c3_ref, out_ref,
                      *, RD, H, CD):
    upd_in = RD + H + CD
    rf = rf_ref[0]
    counts = jnp.maximum(cnt_ref[0][:, 0:1] + cnt_ref[1][:, 0:1], 1.0)
    a = agg_ref[0] / counts
    cond = cond_ref[pl.ds(pl.program_id(0), 1), :]

    mu = (jnp.sum(rf, axis=1, keepdims=True) + jnp.sum(a, axis=1, keepdims=True)
          + jnp.sum(cond)) / upd_in
    m2 = (jnp.sum(rf * rf, axis=1, keepdims=True)
          + jnp.sum(a * a, axis=1, keepdims=True) + jnp.sum(cond * cond)) / upd_in
    var = m2 - mu * mu
    rstd = lax.rsqrt(var + 1e-5)

    nr = (rf - mu) * rstd * lng_ref[:, 0:RD] + lnb_ref[:, 0:RD]
    na = (a - mu) * rstd * lng_ref[:, RD:RD + H] + lnb_ref[:, RD:RD + H]
    nc = (cond - mu) * rstd * lng_ref[:, RD + H:] + lnb_ref[:, RD + H:]

    h = jnp.dot(nr, u1_ref[0:RD, :], preferred_element_type=jnp.float32)
    h += jnp.dot(na, u1_ref[RD:RD + H, :], preferred_element_type=jnp.float32)
    h += jnp.dot(nc, u1_ref[RD + H:, :], preferred_element_type=jnp.float32)
    h = jnp.maximum(h + c1_ref[...], 0.0)
    h = jnp.maximum(
        jnp.dot(h, u2_ref[...], preferred_element_type=jnp.float32) + c2_ref[...], 0.0)
    out_ref[0] = rf + jnp.dot(h, u3_ref[...], preferred_element_type=jnp.float32) + c3_ref[...]


def _node_update(receiver_features, aggsum, cnt_t, conditioning, lng, lnb,
                 u1, c1, u2, c2, u3, c3, bn):
    B, NR, RD = receiver_features.shape
    H = aggsum.shape[2]
    CD = conditioning.shape[1]
    upd_in = RD + H + CD
    grid = (B, NR // bn)
    body = functools.partial(_node_update_body, RD=RD, H=H, CD=CD)
    return pl.pallas_call(
        body,
        grid=grid,
        in_specs=[
            pl.BlockSpec((1, bn, RD), lambda b, j: (b, j, 0)),
            pl.BlockSpec((1, bn, H), lambda b, j: (b, j, 0)),
            pl.BlockSpec((2, bn, H), lambda b, j: (0, j, 0)),
            pl.BlockSpec((B, CD), lambda b, j: (0, 0)),
            pl.BlockSpec((1, upd_in), lambda b, j: (0, 0)),
            pl.BlockSpec((1, upd_in), lambda b, j: (0, 0)),
            pl.BlockSpec((upd_in, H), lambda b, j: (0, 0)),
            pl.BlockSpec((1, H), lambda b, j: (0, 0)),
            pl.BlockSpec((H, H), lambda b, j: (0, 0)),
            pl.BlockSpec((1, H), lambda b, j: (0, 0)),
            pl.BlockSpec((H, RD), lambda b, j: (0, 0)),
            pl.BlockSpec((1, RD), lambda b, j: (0, 0)),
        ],
        out_specs=pl.BlockSpec((1, bn, RD), lambda b, j: (b, j, 0)),
        out_shape=jax.ShapeDtypeStruct((B, NR, RD), jnp.float32),
    )(receiver_features, aggsum, cnt_t, conditioning, lng, lnb,
      u1, c1, u2, c2, u3, c3)


def kernel(sender_features, receiver_features, edge_features, senders, receivers,
           conditioning, msg_W1, msg_b1, msg_W2, msg_b2, msg_W3, msg_b3,
           upd_W1, upd_b1, upd_W2, upd_b2, upd_W3, upd_b3, ln_gamma, ln_beta):
    B, NS, SD = sender_features.shape
    _, NR, RD = receiver_features.shape
    E, ED = edge_features.shape
    CD = conditioning.shape[1]
    H = msg_W2.shape[0]
    assert B == 2 and NS == NR and SD == RD
    assert E % (NUM_TILES * SUPG * CHG) == 0 and NR % CHG == 0
    assert E % (NUM_TILES * CHS) == 0 and NR % CHS == 0

    w1s = msg_W1[:SD]
    w1r = msg_W1[SD:SD + RD]
    w1e = msg_W1[SD + RD:SD + RD + ED]
    w1c = msg_W1[SD + RD + ED:]
    b1 = msg_b1.reshape(1, H)
    b2 = msg_b2.reshape(1, H)
    b3 = msg_b3.reshape(1, H)
    c1 = upd_b1.reshape(1, H)
    c2 = upd_b2.reshape(1, H)
    c3 = upd_b3.reshape(1, RD)
    lng = ln_gamma.reshape(1, -1)
    lnb = ln_beta.reshape(1, -1)
    nchg = E // (NUM_TILES * CHG)
    s4d = senders.astype(jnp.int32).reshape(NUM_TILES, nchg, 1, CHG)
    r4d = receivers.astype(jnp.int32).reshape(NUM_TILES, nchg, 1, CHG)
    r3d = receivers.astype(jnp.int32).reshape(NUM_TILES, E // (NUM_TILES * CHS), CHS)

    sp, rp = _node_projections(sender_features, receiver_features, w1s, w1r, bn=2000)

    gather = _make_gather_kernel(B, E, NS, NR, H)
    g, cnt = gather(sp[0], rp[0], sp[1], rp[1], s4d, r4d)
    g = g.reshape(B, E, H)
    cnt = cnt.reshape(2, NR, H)

    msg = _edge_mlp(g, edge_features, conditioning, w1e, w1c, b1,
                    msg_W2, b2, msg_W3, b3, be=2000)

    scatter = _make_scatter_kernel(B, E, NR, H)
    aggsum = scatter(msg.reshape(B * E, H), r3d).reshape(B, NR, H)

    return _node_update(receiver_features, aggsum, cnt, conditioning, lng, lnb,
                        upd_W1, c1, upd_W2, c2, upd_W3, c3, bn=2000)


# speedup vs baseline: 23.0997x; 1.3787x over previous
import functools

import jax
import jax.numpy as jnp
from jax import lax
from jax.experimental import pallas as pl
from jax.experimental.pallas import tpu as pltpu
from jax.experimental.pallas import tpu_sc as plsc

NUM_TILES = 16
CHS = 80
CHG = 40
SUPG = 50


def _proj_body(s_ref, r_ref, ws_ref, wr_ref, sp_ref, rp_ref):
    sp_ref[0] = jnp.dot(s_ref[0], ws_ref[...], preferred_element_type=jnp.float32)
    rp_ref[0] = jnp.dot(r_ref[0], wr_ref[...], preferred_element_type=jnp.float32)


def _node_projections(sender_features, receiver_features, w1s, w1r, bn):
    B, NS, SD = sender_features.shape
    H = w1s.shape[1]
    grid = (B, NS // bn)
    return pl.pallas_call(
        _proj_body,
        grid=grid,
        in_specs=[
            pl.BlockSpec((1, bn, SD), lambda b, j: (b, j, 0)),
            pl.BlockSpec((1, bn, SD), lambda b, j: (b, j, 0)),
            pl.BlockSpec((SD, H), lambda b, j: (0, 0)),
            pl.BlockSpec((SD, H), lambda b, j: (0, 0)),
        ],
        out_specs=[
            pl.BlockSpec((1, bn, H), lambda b, j: (b, j, 0)),
            pl.BlockSpec((1, bn, H), lambda b, j: (b, j, 0)),
        ],
        out_shape=[
            jax.ShapeDtypeStruct((B, NS, H), jnp.float32),
            jax.ShapeDtypeStruct((B, NS, H), jnp.float32),
        ],
    )(sender_features, receiver_features, w1s, w1r)


def _make_gather_kernel(B, E, NS, NR, H):
    nch = E // (NUM_TILES * CHG)
    nsup = nch // SUPG
    npair = SUPG // 2
    nzc = NR // CHG
    zc_per_tile = -(-nzc // NUM_TILES)
    per_tile = nch * CHG
    mesh = plsc.VectorSubcoreMesh(core_axis_name="c", subcore_axis_name="s")

    @functools.partial(
        pl.kernel,
        mesh=mesh,
        out_type=(
            jax.ShapeDtypeStruct((B * E, H), jnp.float32),
            jax.ShapeDtypeStruct((2 * NR, H), jnp.float32),
        ),
        scratch_types=[
            pltpu.VMEM((SUPG, 1, CHG), jnp.int32),
            pltpu.VMEM((SUPG, 1, CHG), jnp.int32),
            pltpu.VMEM((CHG, H), jnp.float32),
            pltpu.VMEM((CHG, H), jnp.float32),
            pltpu.VMEM((CHG, H), jnp.float32),
            pltpu.VMEM((CHG, H), jnp.float32),
            pltpu.VMEM((CHG, H), jnp.float32),
            pltpu.VMEM_SHARED((NR, H), jnp.float32),
            pltpu.SemaphoreType.DMA,
            pltpu.SemaphoreType.DMA,
            pltpu.SemaphoreType.DMA,
            pltpu.SemaphoreType.DMA,
        ],
    )
    def gather_kernel(sp0, rp0, sp1, rp1, s4d, r4d, g_hbm, cnt_hbm,
                      sidx, ridx, sa, ra, sb, rb, obuf, cacc,
                      sem_a, sem_b, sem_wa, sem_wb):
        c = lax.axis_index("c")
        s = lax.axis_index("s")

        def zrow(i, _):
            for k in range(H // 16):
                obuf[i, pl.ds(k * 16, 16)] = jnp.zeros((16,), jnp.float32)
            return None
        lax.fori_loop(0, CHG, zrow, None)

        def zchunk(t, _):
            j = s + t * NUM_TILES

            @pl.when(j < nzc)
            def _():
                pltpu.sync_copy(obuf, cacc.at[pl.ds(j * CHG, CHG)])
            return None
        lax.fori_loop(0, zc_per_tile, zchunk, None)

        def orow(i, _):
            obuf[i, pl.ds(0, 16)] = jnp.ones((16,), jnp.float32)
            return None
        lax.fori_loop(0, CHG, orow, None)
        plsc.subcore_barrier()

        def run(sp, rp):
            def gissue(jj, sbuf, rbuf, sem):
                pltpu.async_copy(sp.at[sidx.at[jj, 0]], sbuf, sem)
                pltpu.async_copy(rp.at[ridx.at[jj, 0]], rbuf, sem)

            def gwait(sbuf, rbuf, sem):
                pltpu.make_async_copy(sp.at[sidx.at[0, 0]], sbuf, sem).wait()
                pltpu.make_async_copy(rp.at[ridx.at[0, 0]], rbuf, sem).wait()

            def add(sbuf, rbuf):
                @plsc.parallel_loop(0, CHG, unroll=2)
                def _(i):
                    for k in range(H // 16):
                        sl = pl.ds(k * 16, 16)
                        sbuf[i, sl] = sbuf[i, sl] + rbuf[i, sl]

            def wstart(jabs, sbuf, sem):
                e0 = c * E + s * per_tile + jabs * CHG
                pltpu.async_copy(sbuf, g_hbm.at[pl.ds(e0, CHG)], sem)

            def wwait(sbuf, sem):
                pltpu.make_async_copy(sbuf, g_hbm.at[pl.ds(0, CHG)], sem).wait()

            def sup(u, _):
                pltpu.sync_copy(s4d.at[s, pl.ds(u * SUPG, SUPG)], sidx)
                pltpu.sync_copy(r4d.at[s, pl.ds(u * SUPG, SUPG)], ridx)
                gissue(0, sa, ra, sem_a)
                gissue(1, sb, rb, sem_b)
                base = u * SUPG

                def pair(p, _):
                    gwait(sa, ra, sem_a)
                    add(sa, ra)
                    wstart(base + 2 * p, sa, sem_wa)
                    gwait(sb, rb, sem_b)
                    add(sb, rb)
                    wstart(base + 2 * p + 1, sb, sem_wb)
                    wwait(sa, sem_wa)

                    @pl.when(p < npair - 1)
                    def _():
                        gissue(2 * p + 2, sa, ra, sem_a)
                    wwait(sb, sem_wb)

                    @pl.when(p < npair - 1)
                    def _():
                        gissue(2 * p + 3, sb, rb, sem_b)

                    pltpu.sync_copy(obuf, cacc.at[ridx.at[2 * p + c, 0]], add=True)
                    return None

                lax.fori_loop(0, npair, pair, None)
                return None

            lax.fori_loop(0, nsup, sup, None)

        @pl.when(c == 0)
        def _():
            run(sp0, rp0)

        @pl.when(c == 1)
        def _():
            run(sp1, rp1)

        plsc.subcore_barrier()

        def wchunk(t, _):
            j = s + t * NUM_TILES

            @pl.when(j < nzc)
            def _():
                pltpu.sync_copy(cacc.at[pl.ds(j * CHG, CHG)], sa)
                pltpu.sync_copy(sa, cnt_hbm.at[pl.ds(c * NR + j * CHG, CHG)])
            return None
        lax.fori_loop(0, zc_per_tile, wchunk, None)

    return gather_kernel


def _edge_mlp_body(g_ref, ef_ref, cond_ref, w1e_ref, w1c_ref, b1_ref,
                   w2_ref, b2_ref, w3_ref, b3_ref, msg_ref):
    ep = jnp.dot(ef_ref[...], w1e_ref[...], preferred_element_type=jnp.float32)
    cond_row = cond_ref[pl.ds(pl.program_id(0), 1), :]
    cp = jnp.dot(cond_row, w1c_ref[...], preferred_element_type=jnp.float32)
    h1 = jnp.maximum(g_ref[0] + ep + cp + b1_ref[...], 0.0)
    h2 = jnp.maximum(
        jnp.dot(h1, w2_ref[...], preferred_element_type=jnp.float32) + b2_ref[...], 0.0)
    msg_ref[0] = jnp.dot(h2, w3_ref[...], preferred_element_type=jnp.float32) + b3_ref[...]


def _edge_mlp(g, edge_features, conditioning, w1e, w1c, b1, w2, b2, w3, b3, be):
    B, E, H = g.shape
    ED = edge_features.shape[1]
    CD = conditioning.shape[1]
    grid = (B, E // be)
    return pl.pallas_call(
        _edge_mlp_body,
        grid=grid,
        in_specs=[
            pl.BlockSpec((1, be, H), lambda b, j: (b, j, 0)),
            pl.BlockSpec((be, ED), lambda b, j: (j, 0)),
            pl.BlockSpec((B, CD), lambda b, j: (0, 0)),
            pl.BlockSpec((ED, H), lambda b, j: (0, 0)),
            pl.BlockSpec((CD, H), lambda b, j: (0, 0)),
            pl.BlockSpec((1, H), lambda b, j: (0, 0)),
            pl.BlockSpec((H, H), lambda b, j: (0, 0)),
            pl.BlockSpec((1, H), lambda b, j: (0, 0)),
            pl.BlockSpec((H, H), lambda b, j: (0, 0)),
            pl.BlockSpec((1, H), lambda b, j: (0, 0)),
        ],
        out_specs=pl.BlockSpec((1, be, H), lambda b, j: (b, j, 0)),
        out_shape=jax.ShapeDtypeStruct((B, E, H), jnp.float32),
    )(g, edge_features, conditioning, w1e, w1c, b1, w2, b2, w3, b3)


def _make_scatter_kernel(B, E, NR, H):
    nch = E // (NUM_TILES * CHS)
    nzc = NR // CHS
    zc_per_tile = -(-nzc // NUM_TILES)
    mesh = plsc.VectorSubcoreMesh(core_axis_name="c", subcore_axis_name="s")

    @functools.partial(
        pl.kernel,
        mesh=mesh,
        out_type=jax.ShapeDtypeStruct((B * NR, H), jnp.float32),
        scratch_types=[
            pltpu.VMEM((nch, CHS), jnp.int32),
            pltpu.VMEM((CHS, H), jnp.float32),
            pltpu.VMEM((CHS, H), jnp.float32),
            pltpu.VMEM_SHARED((NR, H), jnp.float32),
            pltpu.SemaphoreType.DMA,
            pltpu.SemaphoreType.DMA,
            pltpu.SemaphoreType.DMA,
            pltpu.SemaphoreType.DMA,
        ],
    )
    def scatter_kernel(msg_hbm, r3d, agg_hbm, ridx, ma, mb,
                       acc, sem_la, sem_lb, sem_sa, sem_sb):
        c = lax.axis_index("c")
        s = lax.axis_index("s")
        npair = nch // 2

        def zrow(i, _):
            for k in range(H // 16):
                ma[i, pl.ds(k * 16, 16)] = jnp.zeros((16,), jnp.float32)
            return None
        lax.fori_loop(0, CHS, zrow, None)

        def zchunk(t, _):
            j = s + t * NUM_TILES

            @pl.when(j < nzc)
            def _():
                pltpu.sync_copy(ma, acc.at[pl.ds(j * CHS, CHS)])
            return None
        lax.fori_loop(0, zc_per_tile, zchunk, None)
        plsc.subcore_barrier()

        pltpu.sync_copy(r3d.at[s], ridx)
        base = c * E + s * (nch * CHS)

        def lstart(j, buf, sem):
            pltpu.async_copy(msg_hbm.at[pl.ds(base + j * CHS, CHS)], buf, sem)

        def lwait(buf, sem):
            pltpu.make_async_copy(msg_hbm.at[pl.ds(base, CHS)], buf, sem).wait()

        def sstart(j, buf, sem):
            pltpu.async_copy(buf, acc.at[ridx.at[j]], sem, add=True)

        def swait(buf, sem):
            pltpu.make_async_copy(buf, acc.at[ridx.at[0]], sem).wait()

        lstart(0, ma, sem_la)
        lstart(1, mb, sem_lb)

        def pair(p, _):
            lwait(ma, sem_la)
            sstart(2 * p, ma, sem_sa)
            lwait(mb, sem_lb)
            sstart(2 * p + 1, mb, sem_sb)
            swait(ma, sem_sa)
            pltpu.async_copy(msg_hbm.at[pl.ds(base + (2 * p + 2) * CHS, CHS)],
                             ma, sem_la)
            swait(mb, sem_sb)

            @pl.when(p < npair - 1)
            def _():
                lstart(2 * p + 3, mb, sem_lb)
            return None
        lax.fori_loop(0, npair, pair, None)

        lwait(ma, sem_la)
        sstart(nch - 1, ma, sem_sa)
        swait(ma, sem_sa)
        plsc.subcore_barrier()

        def wchunk(t, _):
            j = s + t * NUM_TILES

            @pl.when(j < nzc)
            def _():
                pltpu.sync_copy(acc.at[pl.ds(j * CHS, CHS)], ma)
                pltpu.sync_copy(ma, agg_hbm.at[pl.ds(c * NR + j * CHS, CHS)])
            return None
        lax.fori_loop(0, zc_per_tile, wchunk, None)

    return scatter_kernel


def _node_update_body(rf_ref, agg_ref, cnt_ref, cond_ref, lng_ref, lnb_ref,
                      u1_ref, c1_ref, u2_ref, c2_ref, u3_ref, c3_ref, out_ref,
                      *, RD, H, CD):
    upd_in = RD + H + CD
    rf = rf_ref[0]
    counts = jnp.maximum(cnt_ref[0][:, 0:1] + cnt_ref[1][:, 0:1], 1.0)
    a = agg_ref[0] / counts
    cond = cond_ref[pl.ds(pl.program_id(0), 1), :]

    mu = (jnp.sum(rf, axis=1, keepdims=True) + jnp.sum(a, axis=1, keepdims=True)
          + jnp.sum(cond)) / upd_in
    m2 = (jnp.sum(rf * rf, axis=1, keepdims=True)
          + jnp.sum(a * a, axis=1, keepdims=True) + jnp.sum(cond * cond)) / upd_in
    var = m2 - mu * mu
    rstd = lax.rsqrt(var + 1e-5)

    nr = (rf - mu) * rstd * lng_ref[:, 0:RD] + lnb_ref[:, 0:RD]
    na = (a - mu) * rstd * lng_ref[:, RD:RD + H] + lnb_ref[:, RD:RD + H]
    nc = (cond - mu) * rstd * lng_ref[:, RD + H:] + lnb_ref[:, RD + H:]

    h = jnp.dot(nr, u1_ref[0:RD, :], preferred_element_type=jnp.float32)
    h += jnp.dot(na, u1_ref[RD:RD + H, :], preferred_element_type=jnp.float32)
    h += jnp.dot(nc, u1_ref[RD + H:, :], preferred_element_type=jnp.float32)
    h = jnp.maximum(h + c1_ref[...], 0.0)
    h = jnp.maximum(
        jnp.dot(h, u2_ref[...], preferred_element_type=jnp.float32) + c2_ref[...], 0.0)
    out_ref[0] = rf + jnp.dot(h, u3_ref[...], preferred_element_type=jnp.float32) + c3_ref[...]


def _node_update(receiver_features, aggsum, cnt_t, conditioning, lng, lnb,
                 u1, c1, u2, c2, u3, c3, bn):
    B, NR, RD = receiver_features.shape
    H = aggsum.shape[2]
    CD = conditioning.shape[1]
    upd_in = RD + H + CD
    grid = (B, NR // bn)
    body = functools.partial(_node_update_body, RD=RD, H=H, CD=CD)
    return pl.pallas_call(
        body,
        grid=grid,
        in_specs=[
            pl.BlockSpec((1, bn, RD), lambda b, j: (b, j, 0)),
            pl.BlockSpec((1, bn, H), lambda b, j: (b, j, 0)),
            pl.BlockSpec((2, bn, H), lambda b, j: (0, j, 0)),
            pl.BlockSpec((B, CD), lambda b, j: (0, 0)),
            pl.BlockSpec((1, upd_in), lambda b, j: (0, 0)),
            pl.BlockSpec((1, upd_in), lambda b, j: (0, 0)),
            pl.BlockSpec((upd_in, H), lambda b, j: (0, 0)),
            pl.BlockSpec((1, H), lambda b, j: (0, 0)),
            pl.BlockSpec((H, H), lambda b, j: (0, 0)),
            pl.BlockSpec((1, H), lambda b, j: (0, 0)),
            pl.BlockSpec((H, RD), lambda b, j: (0, 0)),
            pl.BlockSpec((1, RD), lambda b, j: (0, 0)),
        ],
        out_specs=pl.BlockSpec((1, bn, RD), lambda b, j: (b, j, 0)),
        out_shape=jax.ShapeDtypeStruct((B, NR, RD), jnp.float32),
    )(receiver_features, aggsum, cnt_t, conditioning, lng, lnb,
      u1, c1, u2, c2, u3, c3)


def kernel(sender_features, receiver_features, edge_features, senders, receivers,
           conditioning, msg_W1, msg_b1, msg_W2, msg_b2, msg_W3, msg_b3,
           upd_W1, upd_b1, upd_W2, upd_b2, upd_W3, upd_b3, ln_gamma, ln_beta):
    B, NS, SD = sender_features.shape
    _, NR, RD = receiver_features.shape
    E, ED = edge_features.shape
    CD = conditioning.shape[1]
    H = msg_W2.shape[0]
    assert B == 2 and NS == NR and SD == RD
    assert E % (NUM_TILES * SUPG * CHG) == 0 and NR % CHG == 0
    assert E % (NUM_TILES * CHS) == 0 and NR % CHS == 0

    w1s = msg_W1[:SD]
    w1r = msg_W1[SD:SD + RD]
    w1e = msg_W1[SD + RD:SD + RD + ED]
    w1c = msg_W1[SD + RD + ED:]
    b1 = msg_b1.reshape(1, H)
    b2 = msg_b2.reshape(1, H)
    b3 = msg_b3.reshape(1, H)
    c1 = upd_b1.reshape(1, H)
    c2 = upd_b2.reshape(1, H)
    c3 = upd_b3.reshape(1, RD)
    lng = ln_gamma.reshape(1, -1)
    lnb = ln_beta.reshape(1, -1)
    nchg = E // (NUM_TILES * CHG)
    s4d = senders.astype(jnp.int32).reshape(NUM_TILES, nchg, 1, CHG)
    r4d = receivers.astype(jnp.int32).reshape(NUM_TILES, nchg, 1, CHG)
    r3d = receivers.astype(jnp.int32).reshape(NUM_TILES, E // (NUM_TILES * CHS), CHS)

    sp, rp = _node_projections(sender_features, receiver_features, w1s, w1r, bn=2000)

    gather = _make_gather_kernel(B, E, NS, NR, H)
    g, cnt = gather(sp[0], rp[0], sp[1], rp[1], s4d, r4d)
    g = g.reshape(B, E, H)
    cnt = cnt.reshape(2, NR, H)

    msg = _edge_mlp(g, edge_features, conditioning, w1e, w1c, b1,
                    msg_W2, b2, msg_W3, b3, be=2000)

    scatter = _make_scatter_kernel(B, E, NR, H)
    aggsum = scatter(msg.reshape(B * E, H), r3d).reshape(B, NR, H)

    return _node_update(receiver_features, aggsum, cnt, conditioning, lng, lnb,
                        upd_W1, c1, upd_W2, c2, upd_W3, c3, bn=2000)


# speedup vs baseline: 25.0431x; 1.0841x over previous
import functools

import jax
import jax.numpy as jnp
from jax import lax
from jax.experimental import pallas as pl
from jax.experimental.pallas import tpu as pltpu
from jax.experimental.pallas import tpu_sc as plsc

NUM_TILES = 16
CHS = 80
CHG = 40
SUPG = 50


def _proj_body(s_ref, r_ref, ws_ref, wr_ref, sp_ref, rp_ref):
    sp_ref[0] = jnp.dot(s_ref[0], ws_ref[...], preferred_element_type=jnp.float32)
    rp_ref[0] = jnp.dot(r_ref[0], wr_ref[...], preferred_element_type=jnp.float32)


def _node_projections(sender_features, receiver_features, w1s, w1r, bn):
    B, NS, SD = sender_features.shape
    H = w1s.shape[1]
    grid = (B, NS // bn)
    return pl.pallas_call(
        _proj_body,
        grid=grid,
        in_specs=[
            pl.BlockSpec((1, bn, SD), lambda b, j: (b, j, 0)),
            pl.BlockSpec((1, bn, SD), lambda b, j: (b, j, 0)),
            pl.BlockSpec((SD, H), lambda b, j: (0, 0)),
            pl.BlockSpec((SD, H), lambda b, j: (0, 0)),
        ],
        out_specs=[
            pl.BlockSpec((1, bn, H), lambda b, j: (b, j, 0)),
            pl.BlockSpec((1, bn, H), lambda b, j: (b, j, 0)),
        ],
        out_shape=[
            jax.ShapeDtypeStruct((B, NS, H), jnp.float32),
            jax.ShapeDtypeStruct((B, NS, H), jnp.float32),
        ],
    )(sender_features, receiver_features, w1s, w1r)


def _make_gather_kernel(B, E, NS, NR, H):
    nch = E // (NUM_TILES * CHG)
    nsup = nch // SUPG
    npair = SUPG // 2
    nzc = NR // CHG
    zc_per_tile = -(-nzc // NUM_TILES)
    per_tile = nch * CHG
    mesh = plsc.VectorSubcoreMesh(core_axis_name="c", subcore_axis_name="s")

    @functools.partial(
        pl.kernel,
        mesh=mesh,
        out_type=(
            jax.ShapeDtypeStruct((B * E, H), jnp.float32),
            jax.ShapeDtypeStruct((2 * NR, H), jnp.float32),
        ),
        scratch_types=[
            pltpu.VMEM((SUPG, 1, CHG), jnp.int32),
            pltpu.VMEM((SUPG, 1, CHG), jnp.int32),
            pltpu.VMEM((CHG, H), jnp.float32),
            pltpu.VMEM((CHG, H), jnp.float32),
            pltpu.VMEM((CHG, H), jnp.float32),
            pltpu.VMEM((CHG, H), jnp.float32),
            pltpu.VMEM((CHG, H), jnp.float32),
            pltpu.VMEM_SHARED((NR, H), jnp.float32),
            pltpu.SemaphoreType.DMA,
            pltpu.SemaphoreType.DMA,
            pltpu.SemaphoreType.DMA,
            pltpu.SemaphoreType.DMA,
        ],
    )
    def gather_kernel(sp0, rp0, sp1, rp1, s4d, r4d, g_hbm, cnt_hbm,
                      sidx, ridx, sa, ra, sb, rb, obuf, cacc,
                      sem_a, sem_b, sem_wa, sem_wb):
        c = lax.axis_index("c")
        s = lax.axis_index("s")

        def zrow(i, _):
            for k in range(H // 16):
                obuf[i, pl.ds(k * 16, 16)] = jnp.zeros((16,), jnp.float32)
            return None
        lax.fori_loop(0, CHG, zrow, None)

        def zchunk(t, _):
            j = s + t * NUM_TILES

            @pl.when(j < nzc)
            def _():
                pltpu.sync_copy(obuf, cacc.at[pl.ds(j * CHG, CHG)])
            return None
        lax.fori_loop(0, zc_per_tile, zchunk, None)

        def orow(i, _):
            obuf[i, pl.ds(0, 16)] = jnp.ones((16,), jnp.float32)
            return None
        lax.fori_loop(0, CHG, orow, None)
        plsc.subcore_barrier()

        def run(sp, rp):
            def gissue(jj, sbuf, rbuf, sem):
                pltpu.async_copy(sp.at[sidx.at[jj, 0]], sbuf, sem)
                pltpu.async_copy(rp.at[ridx.at[jj, 0]], rbuf, sem)

            def gwait(sbuf, rbuf, sem):
                pltpu.make_async_copy(sp.at[sidx.at[0, 0]], sbuf, sem).wait()
                pltpu.make_async_copy(rp.at[ridx.at[0, 0]], rbuf, sem).wait()

            def add(sbuf, rbuf):
                @plsc.parallel_loop(0, CHG, unroll=2)
                def _(i):
                    for k in range(H // 16):
                        sl = pl.ds(k * 16, 16)
                        sbuf[i, sl] = sbuf[i, sl] + rbuf[i, sl]

            def wstart(jabs, sbuf, sem):
                e0 = c * E + s * per_tile + jabs * CHG
                pltpu.async_copy(sbuf, g_hbm.at[pl.ds(e0, CHG)], sem)

            def wwait(sbuf, sem):
                pltpu.make_async_copy(sbuf, g_hbm.at[pl.ds(0, CHG)], sem).wait()

            def sup(u, _):
                pltpu.sync_copy(s4d.at[s, pl.ds(u * SUPG, SUPG)], sidx)
                pltpu.sync_copy(r4d.at[s, pl.ds(u * SUPG, SUPG)], ridx)
                gissue(0, sa, ra, sem_a)
                gissue(1, sb, rb, sem_b)
                base = u * SUPG

                def pair(p, _):
                    gwait(sa, ra, sem_a)
                    add(sa, ra)
                    wstart(base + 2 * p, sa, sem_wa)
                    gwait(sb, rb, sem_b)
                    add(sb, rb)
                    wstart(base + 2 * p + 1, sb, sem_wb)
                    wwait(sa, sem_wa)

                    @pl.when(p < npair - 1)
                    def _():
                        gissue(2 * p + 2, sa, ra, sem_a)
                    wwait(sb, sem_wb)

                    @pl.when(p < npair - 1)
                    def _():
                        gissue(2 * p + 3, sb, rb, sem_b)

                    pltpu.sync_copy(obuf, cacc.at[ridx.at[2 * p + c, 0]], add=True)
                    return None

                lax.fori_loop(0, npair, pair, None)
                return None

            lax.fori_loop(0, nsup, sup, None)

        @pl.when(c == 0)
        def _():
            run(sp0, rp0)

        @pl.when(c == 1)
        def _():
            run(sp1, rp1)

        plsc.subcore_barrier()

        def wchunk(t, _):
            j = s + t * NUM_TILES

            @pl.when(j < nzc)
            def _():
                pltpu.sync_copy(cacc.at[pl.ds(j * CHG, CHG)], sa)
                pltpu.sync_copy(sa, cnt_hbm.at[pl.ds(c * NR + j * CHG, CHG)])
            return None
        lax.fori_loop(0, zc_per_tile, wchunk, None)

    return gather_kernel


def _edge_mlp_body(g_ref, ef_ref, cond_ref, w1e_ref, w1c_ref, b1_ref,
                   w2_ref, b2_ref, w3_ref, b3_ref, msg_ref):
    ep = jnp.dot(ef_ref[...], w1e_ref[...], preferred_element_type=jnp.float32)
    cond_row = cond_ref[pl.ds(pl.program_id(0), 1), :]
    cp = jnp.dot(cond_row, w1c_ref[...], preferred_element_type=jnp.float32)
    h1 = jnp.maximum(g_ref[0] + ep + cp + b1_ref[...], 0.0)
    h2 = jnp.maximum(
        jnp.dot(h1, w2_ref[...], preferred_element_type=jnp.float32) + b2_ref[...], 0.0)
    msg_ref[0] = jnp.dot(h2, w3_ref[...], preferred_element_type=jnp.float32) + b3_ref[...]


def _edge_mlp(g, edge_features, conditioning, w1e, w1c, b1, w2, b2, w3, b3, be):
    B, E, H = g.shape
    ED = edge_features.shape[1]
    CD = conditioning.shape[1]
    grid = (B, E // be)
    return pl.pallas_call(
        _edge_mlp_body,
        grid=grid,
        in_specs=[
            pl.BlockSpec((1, be, H), lambda b, j: (b, j, 0)),
            pl.BlockSpec((be, ED), lambda b, j: (j, 0)),
            pl.BlockSpec((B, CD), lambda b, j: (0, 0)),
            pl.BlockSpec((ED, H), lambda b, j: (0, 0)),
            pl.BlockSpec((CD, H), lambda b, j: (0, 0)),
            pl.BlockSpec((1, H), lambda b, j: (0, 0)),
            pl.BlockSpec((H, H), lambda b, j: (0, 0)),
            pl.BlockSpec((1, H), lambda b, j: (0, 0)),
            pl.BlockSpec((H, H), lambda b, j: (0, 0)),
            pl.BlockSpec((1, H), lambda b, j: (0, 0)),
        ],
        out_specs=pl.BlockSpec((1, be, H), lambda b, j: (b, j, 0)),
        out_shape=jax.ShapeDtypeStruct((B, E, H), jnp.float32),
    )(g, edge_features, conditioning, w1e, w1c, b1, w2, b2, w3, b3)


def _make_scatter_kernel(B, E, NR, H):
    nch = E // (NUM_TILES * CHS)
    nzc = NR // CHS
    zc_per_tile = -(-nzc // NUM_TILES)
    mesh = plsc.VectorSubcoreMesh(core_axis_name="c", subcore_axis_name="s")

    @functools.partial(
        pl.kernel,
        mesh=mesh,
        out_type=jax.ShapeDtypeStruct((B * NR, H), jnp.float32),
        scratch_types=[
            pltpu.VMEM((nch, CHS), jnp.int32),
            pltpu.VMEM((CHS, H), jnp.float32),
            pltpu.VMEM((CHS, H), jnp.float32),
            pltpu.VMEM_SHARED((NR, H), jnp.float32),
            pltpu.SemaphoreType.DMA,
            pltpu.SemaphoreType.DMA,
            pltpu.SemaphoreType.DMA,
            pltpu.SemaphoreType.DMA,
        ],
    )
    def scatter_kernel(msg_hbm, r3d, agg_hbm, ridx, ma, mb,
                       acc, sem_la, sem_lb, sem_sa, sem_sb):
        c = lax.axis_index("c")
        s = lax.axis_index("s")
        npair = nch // 2

        def zrow(i, _):
            for k in range(H // 16):
                ma[i, pl.ds(k * 16, 16)] = jnp.zeros((16,), jnp.float32)
            return None
        lax.fori_loop(0, CHS, zrow, None)

        def zchunk(t, _):
            j = s + t * NUM_TILES

            @pl.when(j < nzc)
            def _():
                pltpu.sync_copy(ma, acc.at[pl.ds(j * CHS, CHS)])
            return None
        lax.fori_loop(0, zc_per_tile, zchunk, None)
        plsc.subcore_barrier()

        pltpu.sync_copy(r3d.at[s], ridx)
        base = c * E + s * (nch * CHS)

        def lstart(j, buf, sem):
            pltpu.async_copy(msg_hbm.at[pl.ds(base + j * CHS, CHS)], buf, sem)

        def lwait(buf, sem):
            pltpu.make_async_copy(msg_hbm.at[pl.ds(base, CHS)], buf, sem).wait()

        def sstart(j, buf, sem):
            pltpu.async_copy(buf, acc.at[ridx.at[j]], sem, add=True)

        def swait(buf, sem):
            pltpu.make_async_copy(buf, acc.at[ridx.at[0]], sem).wait()

        lstart(0, ma, sem_la)
        lstart(1, mb, sem_lb)

        def pair(p, _):
            lwait(ma, sem_la)
            sstart(2 * p, ma, sem_sa)
            lwait(mb, sem_lb)
            sstart(2 * p + 1, mb, sem_sb)
            swait(ma, sem_sa)
            pltpu.async_copy(msg_hbm.at[pl.ds(base + (2 * p + 2) * CHS, CHS)],
                             ma, sem_la)
            swait(mb, sem_sb)

            @pl.when(p < npair - 1)
            def _():
                lstart(2 * p + 3, mb, sem_lb)
            return None
        lax.fori_loop(0, npair, pair, None)

        lwait(ma, sem_la)
        sstart(nch - 1, ma, sem_sa)
        swait(ma, sem_sa)
        plsc.subcore_barrier()

        def wchunk(t, _):
            j = s + t * NUM_TILES

            @pl.when(j < nzc)
            def _():
                pltpu.sync_copy(acc.at[pl.ds(j * CHS, CHS)], ma)
                pltpu.sync_copy(ma, agg_hbm.at[pl.ds(c * NR + j * CHS, CHS)])
            return None
        lax.fori_loop(0, zc_per_tile, wchunk, None)

    return scatter_kernel


def _node_update_body(rf_ref, agg_ref, cnt_ref, cond_ref, lng_ref, lnb_ref,
                      u1_ref, c1_ref, u2_ref, c2_ref, u3_ref, c3_ref, out_ref,
                      *, RD, H, CD):
    upd_in = RD + H + CD
    rf = rf_ref[0]
    counts = jnp.maximum(cnt_ref[0][:, 0:1] + cnt_ref[1][:, 0:1], 1.0)
    a = agg_ref[0] / counts
    cond = cond_ref[pl.ds(pl.program_id(0), 1), :]

    mu = (jnp.sum(rf, axis=1, keepdims=True) + jnp.sum(a, axis=1, keepdims=True)
          + jnp.sum(cond)) / upd_in
    m2 = (jnp.sum(rf * rf, axis=1, keepdims=True)
          + jnp.sum(a * a, axis=1, keepdims=True) + jnp.sum(cond * cond)) / upd_in
    var = m2 - mu * mu
    rstd = lax.rsqrt(var + 1e-5)

    nr = (rf - mu) * rstd * lng_ref[:, 0:RD] + lnb_ref[:, 0:RD]
    na = (a - mu) * rstd * lng_ref[:, RD:RD + H] + lnb_ref[:, RD:RD + H]
    nc = (cond - mu) * rstd * lng_ref[:, RD + H:] + lnb_ref[:, RD + H:]

    h = jnp.dot(nr, u1_ref[0:RD, :], preferred_element_type=jnp.float32)
    h += jnp.dot(na, u1_ref[RD:RD + H, :], preferred_element_type=jnp.float32)
    h += jnp.dot(nc, u1_ref[RD + H:, :], preferred_element_type=jnp.float32)
    h = jnp.maximum(h + c1_ref[...], 0.0)
    h = jnp.maximum(
        jnp.dot(h, u2_ref[...], preferred_element_type=jnp.float32) + c2_ref[...], 0.0)
    out_ref[0] = rf + jnp.dot(h, u3_ref[...], preferred_element_type=jnp.float32) + c3_ref[...]


def _node_update(receiver_features, aggsum, cnt_t, conditioning, lng, lnb,
                 u1, c1, u2, c2, u3, c3, bn):
    B, NR, RD = receiver_features.shape
    H = aggsum.shape[2]
    CD = conditioning.shape[1]
    upd_in = RD + H + CD
    grid = (B, NR // bn)
    body = functools.partial(_node_update_body, RD=RD, H=H, CD=CD)
    return pl.pallas_call(
        body,
        grid=grid,
        in_specs=[
            pl.BlockSpec((1, bn, RD), lambda b, j: (b, j, 0)),
            pl.BlockSpec((1, bn, H), lambda b, j: (b, j, 0)),
            pl.BlockSpec((2, bn, H), lambda b, j: (0, j, 0)),
            pl.BlockSpec((B, CD), lambda b, j: (0, 0)),
            pl.BlockSpec((1, upd_in), lambda b, j: (0, 0)),
            pl.BlockSpec((1, upd_in), lambda b, j: (0, 0)),
            pl.BlockSpec((upd_in, H), lambda b, j: (0, 0)),
            pl.BlockSpec((1, H), lambda b, j: (0, 0)),
            pl.BlockSpec((H, H), lambda b, j: (0, 0)),
            pl.BlockSpec((1, H), lambda b, j: (0, 0)),
            pl.BlockSpec((H, RD), lambda b, j: (0, 0)),
            pl.BlockSpec((1, RD), lambda b, j: (0, 0)),
        ],
        out_specs=pl.BlockSpec((1, bn, RD), lambda b, j: (b, j, 0)),
        out_shape=jax.ShapeDtypeStruct((B, NR, RD), jnp.float32),
    )(receiver_features, aggsum, cnt_t, conditioning, lng, lnb,
      u1, c1, u2, c2, u3, c3)


def kernel(sender_features, receiver_features, edge_features, senders, receivers,
           conditioning, msg_W1, msg_b1, msg_W2, msg_b2, msg_W3, msg_b3,
           upd_W1, upd_b1, upd_W2, upd_b2, upd_W3, upd_b3, ln_gamma, ln_beta):
    B, NS, SD = sender_features.shape
    _, NR, RD = receiver_features.shape
    E, ED = edge_features.shape
    CD = conditioning.shape[1]
    H = msg_W2.shape[0]
    assert B == 2 and NS == NR and SD == RD
    assert E % (NUM_TILES * SUPG * CHG) == 0 and NR % CHG == 0
    assert E % (NUM_TILES * CHS) == 0 and NR % CHS == 0

    w1s = msg_W1[:SD]
    w1r = msg_W1[SD:SD + RD]
    w1e = msg_W1[SD + RD:SD + RD + ED]
    w1c = msg_W1[SD + RD + ED:]
    b1 = msg_b1.reshape(1, H)
    b2 = msg_b2.reshape(1, H)
    b3 = msg_b3.reshape(1, H)
    c1 = upd_b1.reshape(1, H)
    c2 = upd_b2.reshape(1, H)
    c3 = upd_b3.reshape(1, RD)
    lng = ln_gamma.reshape(1, -1)
    lnb = ln_beta.reshape(1, -1)
    nchg = E // (NUM_TILES * CHG)
    s4d = senders.astype(jnp.int32).reshape(NUM_TILES, nchg, 1, CHG)
    r4d = receivers.astype(jnp.int32).reshape(NUM_TILES, nchg, 1, CHG)
    r3d = receivers.astype(jnp.int32).reshape(NUM_TILES, E // (NUM_TILES * CHS), CHS)

    sp, rp = _node_projections(sender_features, receiver_features, w1s, w1r, bn=5000)

    gather = _make_gather_kernel(B, E, NS, NR, H)
    g, cnt = gather(sp[0], rp[0], sp[1], rp[1], s4d, r4d)
    g = g.reshape(B, E, H)
    cnt = cnt.reshape(2, NR, H)

    msg = _edge_mlp(g, edge_features, conditioning, w1e, w1c, b1,
                    msg_W2, b2, msg_W3, b3, be=4000)

    scatter = _make_scatter_kernel(B, E, NR, H)
    aggsum = scatter(msg.reshape(B * E, H), r3d).reshape(B, NR, H)

    return _node_update(receiver_features, aggsum, cnt, conditioning, lng, lnb,
                        upd_W1, c1, upd_W2, c2, upd_W3, c3, bn=2000)


# speedup vs baseline: 25.3997x; 1.0142x over previous
import functools

import jax
import jax.numpy as jnp
from jax import lax
from jax.experimental import pallas as pl
from jax.experimental.pallas import tpu as pltpu
from jax.experimental.pallas import tpu_sc as plsc

NUM_TILES = 16
CHS = 80
CHG = 40
SUPG = 50


def _proj_body(s_ref, r_ref, ws_ref, wr_ref, sp_ref, rp_ref):
    sp_ref[0] = jnp.dot(s_ref[0], ws_ref[...], preferred_element_type=jnp.float32)
    rp_ref[0] = jnp.dot(r_ref[0], wr_ref[...], preferred_element_type=jnp.float32)


def _node_projections(sender_features, receiver_features, w1s, w1r, bn):
    B, NS, SD = sender_features.shape
    H = w1s.shape[1]
    grid = (B, NS // bn)
    return pl.pallas_call(
        _proj_body,
        grid=grid,
        in_specs=[
            pl.BlockSpec((1, bn, SD), lambda b, j: (b, j, 0)),
            pl.BlockSpec((1, bn, SD), lambda b, j: (b, j, 0)),
            pl.BlockSpec((SD, H), lambda b, j: (0, 0)),
            pl.BlockSpec((SD, H), lambda b, j: (0, 0)),
        ],
        out_specs=[
            pl.BlockSpec((1, bn, H), lambda b, j: (b, j, 0)),
            pl.BlockSpec((1, bn, H), lambda b, j: (b, j, 0)),
        ],
        out_shape=[
            jax.ShapeDtypeStruct((B, NS, H), jnp.float32),
            jax.ShapeDtypeStruct((B, NS, H), jnp.float32),
        ],
    )(sender_features, receiver_features, w1s, w1r)


def _make_gather_kernel(B, E, NS, NR, H):
    nch = E // (NUM_TILES * CHG)
    nsup = nch // SUPG
    npair = SUPG // 2
    nzc = NR // CHG
    zc_per_tile = -(-nzc // NUM_TILES)
    per_tile = nch * CHG
    mesh = plsc.VectorSubcoreMesh(core_axis_name="c", subcore_axis_name="s")

    @functools.partial(
        pl.kernel,
        mesh=mesh,
        out_type=(
            jax.ShapeDtypeStruct((B * E, H), jnp.float32),
            jax.ShapeDtypeStruct((2 * NR, H), jnp.float32),
        ),
        scratch_types=[
            pltpu.VMEM((SUPG, 1, CHG), jnp.int32),
            pltpu.VMEM((SUPG, 1, CHG), jnp.int32),
            pltpu.VMEM((CHG, H), jnp.float32),
            pltpu.VMEM((CHG, H), jnp.float32),
            pltpu.VMEM((CHG, H), jnp.float32),
            pltpu.VMEM((CHG, H), jnp.float32),
            pltpu.VMEM((CHG, H), jnp.float32),
            pltpu.VMEM_SHARED((NR, H), jnp.float32),
            pltpu.SemaphoreType.DMA,
            pltpu.SemaphoreType.DMA,
            pltpu.SemaphoreType.DMA,
            pltpu.SemaphoreType.DMA,
        ],
    )
    def gather_kernel(sp0, rp0, sp1, rp1, s4d, r4d, g_hbm, cnt_hbm,
                      sidx, ridx, sa, ra, sb, rb, obuf, cacc,
                      sem_a, sem_b, sem_wa, sem_wb):
        c = lax.axis_index("c")
        s = lax.axis_index("s")

        def zrow(i, _):
            for k in range(H // 16):
                obuf[i, pl.ds(k * 16, 16)] = jnp.zeros((16,), jnp.float32)
            return None
        lax.fori_loop(0, CHG, zrow, None)

        def zchunk(t, _):
            j = s + t * NUM_TILES

            @pl.when(j < nzc)
            def _():
                pltpu.sync_copy(obuf, cacc.at[pl.ds(j * CHG, CHG)])
            return None
        lax.fori_loop(0, zc_per_tile, zchunk, None)

        def orow(i, _):
            obuf[i, pl.ds(0, 16)] = jnp.ones((16,), jnp.float32)
            return None
        lax.fori_loop(0, CHG, orow, None)
        plsc.subcore_barrier()

        def run(sp, rp):
            def gissue(jj, sbuf, rbuf, sem):
                pltpu.async_copy(sp.at[sidx.at[jj, 0]], sbuf, sem)
                pltpu.async_copy(rp.at[ridx.at[jj, 0]], rbuf, sem)

            def gwait(sbuf, rbuf, sem):
                pltpu.make_async_copy(sp.at[sidx.at[0, 0]], sbuf, sem).wait()
                pltpu.make_async_copy(rp.at[ridx.at[0, 0]], rbuf, sem).wait()

            def add(sbuf, rbuf):
                @plsc.parallel_loop(0, CHG, unroll=2)
                def _(i):
                    for k in range(H // 16):
                        sl = pl.ds(k * 16, 16)
                        sbuf[i, sl] = sbuf[i, sl] + rbuf[i, sl]

            def wstart(jabs, sbuf, sem):
                e0 = c * E + s * per_tile + jabs * CHG
                pltpu.async_copy(sbuf, g_hbm.at[pl.ds(e0, CHG)], sem)

            def wwait(sbuf, sem):
                pltpu.make_async_copy(sbuf, g_hbm.at[pl.ds(0, CHG)], sem).wait()

            def sup(u, _):
                pltpu.sync_copy(s4d.at[s, pl.ds(u * SUPG, SUPG)], sidx)
                pltpu.sync_copy(r4d.at[s, pl.ds(u * SUPG, SUPG)], ridx)
                gissue(0, sa, ra, sem_a)
                gissue(1, sb, rb, sem_b)
                base = u * SUPG

                def pair(p, _):
                    gwait(sa, ra, sem_a)
                    add(sa, ra)
                    wstart(base + 2 * p, sa, sem_wa)
                    gwait(sb, rb, sem_b)
                    add(sb, rb)
                    wstart(base + 2 * p + 1, sb, sem_wb)
                    wwait(sa, sem_wa)

                    @pl.when(p < npair - 1)
                    def _():
                        gissue(2 * p + 2, sa, ra, sem_a)
                    wwait(sb, sem_wb)

                    @pl.when(p < npair - 1)
                    def _():
                        gissue(2 * p + 3, sb, rb, sem_b)

                    pltpu.sync_copy(obuf, cacc.at[ridx.at[2 * p + c, 0]], add=True)
                    return None

                lax.fori_loop(0, npair, pair, None)
                return None

            lax.fori_loop(0, nsup, sup, None)

        @pl.when(c == 0)
        def _():
            run(sp0, rp0)

        @pl.when(c == 1)
        def _():
            run(sp1, rp1)

        plsc.subcore_barrier()

        def wchunk(t, _):
            j = s + t * NUM_TILES

            @pl.when(j < nzc)
            def _():
                pltpu.sync_copy(cacc.at[pl.ds(j * CHG, CHG)], sa)
                pltpu.sync_copy(sa, cnt_hbm.at[pl.ds(c * NR + j * CHG, CHG)])
            return None
        lax.fori_loop(0, zc_per_tile, wchunk, None)

    return gather_kernel


def _edge_mlp_body(g_ref, ef_ref, cond_ref, w1e_ref, w1c_ref, b1_ref,
                   w2_ref, b2_ref, w3_ref, b3_ref, msg_ref):
    ep = jnp.dot(ef_ref[...], w1e_ref[...], preferred_element_type=jnp.float32)
    cond_row = cond_ref[pl.ds(pl.program_id(0), 1), :]
    cp = jnp.dot(cond_row, w1c_ref[...], preferred_element_type=jnp.float32)
    h1 = jnp.maximum(g_ref[0] + ep + cp + b1_ref[...], 0.0)
    h2 = jnp.maximum(
        jnp.dot(h1, w2_ref[...], preferred_element_type=jnp.float32) + b2_ref[...], 0.0)
    msg_ref[0] = jnp.dot(h2, w3_ref[...], preferred_element_type=jnp.float32) + b3_ref[...]


def _edge_mlp(g, edge_features, conditioning, w1e, w1c, b1, w2, b2, w3, b3, be):
    B, E, H = g.shape
    ED = edge_features.shape[1]
    CD = conditioning.shape[1]
    grid = (B, E // be)
    return pl.pallas_call(
        _edge_mlp_body,
        grid=grid,
        in_specs=[
            pl.BlockSpec((1, be, H), lambda b, j: (b, j, 0)),
            pl.BlockSpec((be, ED), lambda b, j: (j, 0)),
            pl.BlockSpec((B, CD), lambda b, j: (0, 0)),
            pl.BlockSpec((ED, H), lambda b, j: (0, 0)),
            pl.BlockSpec((CD, H), lambda b, j: (0, 0)),
            pl.BlockSpec((1, H), lambda b, j: (0, 0)),
            pl.BlockSpec((H, H), lambda b, j: (0, 0)),
            pl.BlockSpec((1, H), lambda b, j: (0, 0)),
            pl.BlockSpec((H, H), lambda b, j: (0, 0)),
            pl.BlockSpec((1, H), lambda b, j: (0, 0)),
        ],
        out_specs=pl.BlockSpec((1, be, H), lambda b, j: (b, j, 0)),
        out_shape=jax.ShapeDtypeStruct((B, E, H), jnp.float32),
    )(g, edge_features, conditioning, w1e, w1c, b1, w2, b2, w3, b3)


def _make_scatter_kernel(B, E, NR, H):
    nch = E // (NUM_TILES * CHS)
    nzc = NR // CHS
    zc_per_tile = -(-nzc // NUM_TILES)
    mesh = plsc.VectorSubcoreMesh(core_axis_name="c", subcore_axis_name="s")

    @functools.partial(
        pl.kernel,
        mesh=mesh,
        out_type=jax.ShapeDtypeStruct((B * NR, H), jnp.float32),
        scratch_types=[
            pltpu.VMEM((nch, CHS), jnp.int32),
            pltpu.VMEM((CHS, H), jnp.float32),
            pltpu.VMEM((CHS, H), jnp.float32),
            pltpu.VMEM_SHARED((NR, H), jnp.float32),
            pltpu.SemaphoreType.DMA,
            pltpu.SemaphoreType.DMA,
            pltpu.SemaphoreType.DMA,
            pltpu.SemaphoreType.DMA,
        ],
    )
    def scatter_kernel(msg_hbm, r3d, agg_hbm, ridx, ma, mb,
                       acc, sem_la, sem_lb, sem_sa, sem_sb):
        c = lax.axis_index("c")
        s = lax.axis_index("s")
        npair = nch // 2

        def zrow(i, _):
            for k in range(H // 16):
                ma[i, pl.ds(k * 16, 16)] = jnp.zeros((16,), jnp.float32)
            return None
        lax.fori_loop(0, CHS, zrow, None)

        def zchunk(t, _):
            j = s + t * NUM_TILES

            @pl.when(j < nzc)
            def _():
                pltpu.sync_copy(ma, acc.at[pl.ds(j * CHS, CHS)])
            return None
        lax.fori_loop(0, zc_per_tile, zchunk, None)
        plsc.subcore_barrier()

        pltpu.sync_copy(r3d.at[s], ridx)
        base = c * E + s * (nch * CHS)

        def lstart(j, buf, sem):
            pltpu.async_copy(msg_hbm.at[pl.ds(base + j * CHS, CHS)], buf, sem)

        def lwait(buf, sem):
            pltpu.make_async_copy(msg_hbm.at[pl.ds(base, CHS)], buf, sem).wait()

        def sstart(j, buf, sem):
            pltpu.async_copy(buf, acc.at[ridx.at[j]], sem, add=True)

        def swait(buf, sem):
            pltpu.make_async_copy(buf, acc.at[ridx.at[0]], sem).wait()

        lstart(0, ma, sem_la)
        lstart(1, mb, sem_lb)

        def pair(p, _):
            lwait(ma, sem_la)
            sstart(2 * p, ma, sem_sa)
            lwait(mb, sem_lb)
            sstart(2 * p + 1, mb, sem_sb)
            swait(ma, sem_sa)
            pltpu.async_copy(msg_hbm.at[pl.ds(base + (2 * p + 2) * CHS, CHS)],
                             ma, sem_la)
            swait(mb, sem_sb)

            @pl.when(p < npair - 1)
            def _():
                lstart(2 * p + 3, mb, sem_lb)
            return None
        lax.fori_loop(0, npair, pair, None)

        lwait(ma, sem_la)
        sstart(nch - 1, ma, sem_sa)
        swait(ma, sem_sa)
        plsc.subcore_barrier()

        def wchunk(t, _):
            j = s + t * NUM_TILES

            @pl.when(j < nzc)
            def _():
                pltpu.sync_copy(acc.at[pl.ds(j * CHS, CHS)], ma)
                pltpu.sync_copy(ma, agg_hbm.at[pl.ds(c * NR + j * CHS, CHS)])
            return None
        lax.fori_loop(0, zc_per_tile, wchunk, None)

    return scatter_kernel


def _node_update_body(rf_ref, agg_ref, cnt_ref, cond_ref, lng_ref, lnb_ref,
                      u1_ref, c1_ref, u2_ref, c2_ref, u3_ref, c3_ref, out_ref,
                      *, RD, H, CD):
    upd_in = RD + H + CD
    rf = rf_ref[0]
    counts = jnp.maximum(cnt_ref[0][:, 0:1] + cnt_ref[1][:, 0:1], 1.0)
    a = agg_ref[0] / counts
    cond = cond_ref[pl.ds(pl.program_id(0), 1), :]

    mu = (jnp.sum(rf, axis=1, keepdims=True) + jnp.sum(a, axis=1, keepdims=True)
          + jnp.sum(cond)) / upd_in
    m2 = (jnp.sum(rf * rf, axis=1, keepdims=True)
          + jnp.sum(a * a, axis=1, keepdims=True) + jnp.sum(cond * cond)) / upd_in
    var = m2 - mu * mu
    rstd = lax.rsqrt(var + 1e-5)

    nr = (rf - mu) * rstd * lng_ref[:, 0:RD] + lnb_ref[:, 0:RD]
    na = (a - mu) * rstd * lng_ref[:, RD:RD + H] + lnb_ref[:, RD:RD + H]
    nc = (cond - mu) * rstd * lng_ref[:, RD + H:] + lnb_ref[:, RD + H:]

    h = jnp.dot(nr, u1_ref[0:RD, :], preferred_element_type=jnp.float32)
    h += jnp.dot(na, u1_ref[RD:RD + H, :], preferred_element_type=jnp.float32)
    h += jnp.dot(nc, u1_ref[RD + H:, :], preferred_element_type=jnp.float32)
    h = jnp.maximum(h + c1_ref[...], 0.0)
    h = jnp.maximum(
        jnp.dot(h, u2_ref[...], preferred_element_type=jnp.float32) + c2_ref[...], 0.0)
    out_ref[0] = rf + jnp.dot(h, u3_ref[...], preferred_element_type=jnp.float32) + c3_ref[...]


def _node_update(receiver_features, aggsum, cnt_t, conditioning, lng, lnb,
                 u1, c1, u2, c2, u3, c3, bn):
    B, NR, RD = receiver_features.shape
    H = aggsum.shape[2]
    CD = conditioning.shape[1]
    upd_in = RD + H + CD
    grid = (B, NR // bn)
    body = functools.partial(_node_update_body, RD=RD, H=H, CD=CD)
    return pl.pallas_call(
        body,
        grid=grid,
        in_specs=[
            pl.BlockSpec((1, bn, RD), lambda b, j: (b, j, 0)),
            pl.BlockSpec((1, bn, H), lambda b, j: (b, j, 0)),
            pl.BlockSpec((2, bn, H), lambda b, j: (0, j, 0)),
            pl.BlockSpec((B, CD), lambda b, j: (0, 0)),
            pl.BlockSpec((1, upd_in), lambda b, j: (0, 0)),
            pl.BlockSpec((1, upd_in), lambda b, j: (0, 0)),
            pl.BlockSpec((upd_in, H), lambda b, j: (0, 0)),
            pl.BlockSpec((1, H), lambda b, j: (0, 0)),
            pl.BlockSpec((H, H), lambda b, j: (0, 0)),
            pl.BlockSpec((1, H), lambda b, j: (0, 0)),
            pl.BlockSpec((H, RD), lambda b, j: (0, 0)),
            pl.BlockSpec((1, RD), lambda b, j: (0, 0)),
        ],
        out_specs=pl.BlockSpec((1, bn, RD), lambda b, j: (b, j, 0)),
        out_shape=jax.ShapeDtypeStruct((B, NR, RD), jnp.float32),
    )(receiver_features, aggsum, cnt_t, conditioning, lng, lnb,
      u1, c1, u2, c2, u3, c3)


def kernel(sender_features, receiver_features, edge_features, senders, receivers,
           conditioning, msg_W1, msg_b1, msg_W2, msg_b2, msg_W3, msg_b3,
           upd_W1, upd_b1, upd_W2, upd_b2, upd_W3, upd_b3, ln_gamma, ln_beta):
    B, NS, SD = sender_features.shape
    _, NR, RD = receiver_features.shape
    E, ED = edge_features.shape
    CD = conditioning.shape[1]
    H = msg_W2.shape[0]
    assert B == 2 and NS == NR and SD == RD
    assert E % (NUM_TILES * SUPG * CHG) == 0 and NR % CHG == 0
    assert E % (NUM_TILES * CHS) == 0 and NR % CHS == 0

    w1s = msg_W1[:SD]
    w1r = msg_W1[SD:SD + RD]
    w1e = msg_W1[SD + RD:SD + RD + ED]
    w1c = msg_W1[SD + RD + ED:]
    b1 = msg_b1.reshape(1, H)
    b2 = msg_b2.reshape(1, H)
    b3 = msg_b3.reshape(1, H)
    c1 = upd_b1.reshape(1, H)
    c2 = upd_b2.reshape(1, H)
    c3 = upd_b3.reshape(1, RD)
    lng = ln_gamma.reshape(1, -1)
    lnb = ln_beta.reshape(1, -1)
    nchg = E // (NUM_TILES * CHG)
    s4d = senders.astype(jnp.int32).reshape(NUM_TILES, nchg, 1, CHG)
    r4d = receivers.astype(jnp.int32).reshape(NUM_TILES, nchg, 1, CHG)
    r3d = receivers.astype(jnp.int32).reshape(NUM_TILES, E // (NUM_TILES * CHS), CHS)

    sp, rp = _node_projections(sender_features, receiver_features, w1s, w1r, bn=5000)

    gather = _make_gather_kernel(B, E, NS, NR, H)
    g, cnt = gather(sp[0], rp[0], sp[1], rp[1], s4d, r4d)
    g = g.reshape(B, E, H)
    cnt = cnt.reshape(2, NR, H)

    msg = _edge_mlp(g, edge_features, conditioning, w1e, w1c, b1,
                    msg_W2, b2, msg_W3, b3, be=8000)

    scatter = _make_scatter_kernel(B, E, NR, H)
    aggsum = scatter(msg.reshape(B * E, H), r3d).reshape(B, NR, H)

    return _node_update(receiver_features, aggsum, cnt, conditioning, lng, lnb,
                        upd_W1, c1, upd_W2, c2, upd_W3, c3, bn=5000)


# speedup vs baseline: 25.7941x; 1.0155x over previous
import functools

import jax
import jax.numpy as jnp
from jax import lax
from jax.experimental import pallas as pl
from jax.experimental.pallas import tpu as pltpu
from jax.experimental.pallas import tpu_sc as plsc

NUM_TILES = 16
CHS = 80
CHG = 40
SUPG = 50


def _proj_body(s_ref, r_ref, ws_ref, wr_ref, sp_ref, rp_ref):
    sp_ref[0] = jnp.dot(s_ref[0], ws_ref[...], preferred_element_type=jnp.float32)
    rp_ref[0] = jnp.dot(r_ref[0], wr_ref[...], preferred_element_type=jnp.float32)


def _node_projections(sender_features, receiver_features, w1s, w1r, bn):
    B, NS, SD = sender_features.shape
    H = w1s.shape[1]
    grid = (B, NS // bn)
    return pl.pallas_call(
        _proj_body,
        grid=grid,
        in_specs=[
            pl.BlockSpec((1, bn, SD), lambda b, j: (b, j, 0)),
            pl.BlockSpec((1, bn, SD), lambda b, j: (b, j, 0)),
            pl.BlockSpec((SD, H), lambda b, j: (0, 0)),
            pl.BlockSpec((SD, H), lambda b, j: (0, 0)),
        ],
        out_specs=[
            pl.BlockSpec((1, bn, H), lambda b, j: (b, j, 0)),
            pl.BlockSpec((1, bn, H), lambda b, j: (b, j, 0)),
        ],
        out_shape=[
            jax.ShapeDtypeStruct((B, NS, H), jnp.float32),
            jax.ShapeDtypeStruct((B, NS, H), jnp.float32),
        ],
    )(sender_features, receiver_features, w1s, w1r)


def _make_gather_kernel(B, E, NS, NR, H):
    nch = E // (NUM_TILES * CHG)
    nsup = nch // SUPG
    npair = SUPG // 2
    nzc = NR // CHG
    zc_per_tile = -(-nzc // NUM_TILES)
    per_tile = nch * CHG
    mesh = plsc.VectorSubcoreMesh(core_axis_name="c", subcore_axis_name="s")

    @functools.partial(
        pl.kernel,
        mesh=mesh,
        out_type=(
            jax.ShapeDtypeStruct((B * E, H), jnp.float32),
            jax.ShapeDtypeStruct((2 * NR, H), jnp.float32),
        ),
        scratch_types=[
            pltpu.VMEM((SUPG, 1, CHG), jnp.int32),
            pltpu.VMEM((SUPG, 1, CHG), jnp.int32),
            pltpu.VMEM((CHG, H), jnp.float32),
            pltpu.VMEM((CHG, H), jnp.float32),
            pltpu.VMEM((CHG, H), jnp.float32),
            pltpu.VMEM((CHG, H), jnp.float32),
            pltpu.VMEM((CHG, H), jnp.float32),
            pltpu.VMEM_SHARED((NR, H), jnp.float32),
            pltpu.SemaphoreType.DMA,
            pltpu.SemaphoreType.DMA,
            pltpu.SemaphoreType.DMA,
            pltpu.SemaphoreType.DMA,
        ],
    )
    def gather_kernel(sp0, rp0, sp1, rp1, s4d, r4d, g_hbm, cnt_hbm,
                      sidx, ridx, sa, ra, sb, rb, obuf, cacc,
                      sem_a, sem_b, sem_wa, sem_wb):
        c = lax.axis_index("c")
        s = lax.axis_index("s")

        def zrow(i, _):
            for k in range(H // 16):
                obuf[i, pl.ds(k * 16, 16)] = jnp.zeros((16,), jnp.float32)
            return None
        lax.fori_loop(0, CHG, zrow, None)

        def zchunk(t, _):
            j = s + t * NUM_TILES

            @pl.when(j < nzc)
            def _():
                pltpu.sync_copy(obuf, cacc.at[pl.ds(j * CHG, CHG)])
            return None
        lax.fori_loop(0, zc_per_tile, zchunk, None)

        def orow(i, _):
            obuf[i, pl.ds(0, 16)] = jnp.ones((16,), jnp.float32)
            return None
        lax.fori_loop(0, CHG, orow, None)
        plsc.subcore_barrier()

        def run(sp, rp):
            def gissue(jj, sbuf, rbuf, sem):
                pltpu.async_copy(sp.at[sidx.at[jj, 0]], sbuf, sem)
                pltpu.async_copy(rp.at[ridx.at[jj, 0]], rbuf, sem)

            def gwait(sbuf, rbuf, sem):
                pltpu.make_async_copy(sp.at[sidx.at[0, 0]], sbuf, sem).wait()
                pltpu.make_async_copy(rp.at[ridx.at[0, 0]], rbuf, sem).wait()

            def add(sbuf, rbuf):
                @plsc.parallel_loop(0, CHG, unroll=2)
                def _(i):
                    for k in range(H // 16):
                        sl = pl.ds(k * 16, 16)
                        sbuf[i, sl] = sbuf[i, sl] + rbuf[i, sl]

            def wstart(jabs, sbuf, sem):
                e0 = c * E + s * per_tile + jabs * CHG
                pltpu.async_copy(sbuf, g_hbm.at[pl.ds(e0, CHG)], sem)

            def wwait(sbuf, sem):
                pltpu.make_async_copy(sbuf, g_hbm.at[pl.ds(0, CHG)], sem).wait()

            def sup(u, _):
                pltpu.sync_copy(s4d.at[s, pl.ds(u * SUPG, SUPG)], sidx)
                pltpu.sync_copy(r4d.at[s, pl.ds(u * SUPG, SUPG)], ridx)
                gissue(0, sa, ra, sem_a)
                gissue(1, sb, rb, sem_b)
                base = u * SUPG

                def pair(p, _):
                    gwait(sa, ra, sem_a)
                    add(sa, ra)
                    wstart(base + 2 * p, sa, sem_wa)
                    gwait(sb, rb, sem_b)
                    add(sb, rb)
                    wstart(base + 2 * p + 1, sb, sem_wb)
                    wwait(sa, sem_wa)

                    @pl.when(p < npair - 1)
                    def _():
                        gissue(2 * p + 2, sa, ra, sem_a)
                    wwait(sb, sem_wb)

                    @pl.when(p < npair - 1)
                    def _():
                        gissue(2 * p + 3, sb, rb, sem_b)

                    pltpu.sync_copy(obuf, cacc.at[ridx.at[2 * p + c, 0]], add=True)
                    return None

                lax.fori_loop(0, npair, pair, None)
                return None

            lax.fori_loop(0, nsup, sup, None)

        @pl.when(c == 0)
        def _():
            run(sp0, rp0)

        @pl.when(c == 1)
        def _():
            run(sp1, rp1)

        plsc.subcore_barrier()

        def wchunk(t, _):
            j = s + t * NUM_TILES

            @pl.when(j < nzc)
            def _():
                pltpu.sync_copy(cacc.at[pl.ds(j * CHG, CHG)], sa)
                pltpu.sync_copy(sa, cnt_hbm.at[pl.ds(c * NR + j * CHG, CHG)])
            return None
        lax.fori_loop(0, zc_per_tile, wchunk, None)

    return gather_kernel


def _edge_mlp_body(g_ref, ef_ref, cond_ref, w1e_ref, w1c_ref, b1_ref,
                   w2_ref, b2_ref, w3_ref, b3_ref, msg_ref):
    ep = jnp.dot(ef_ref[...], w1e_ref[...], preferred_element_type=jnp.float32)
    cond_row = cond_ref[pl.ds(pl.program_id(0), 1), :]
    cp = jnp.dot(cond_row, w1c_ref[...], preferred_element_type=jnp.float32)
    h1 = jnp.maximum(g_ref[0] + ep + cp + b1_ref[...], 0.0)
    h2 = jnp.maximum(
        jnp.dot(h1, w2_ref[...], preferred_element_type=jnp.float32) + b2_ref[...], 0.0)
    msg_ref[0] = jnp.dot(h2, w3_ref[...], preferred_element_type=jnp.float32) + b3_ref[...]


def _edge_mlp(g, edge_features, conditioning, w1e, w1c, b1, w2, b2, w3, b3, be):
    B, E, H = g.shape
    ED = edge_features.shape[1]
    CD = conditioning.shape[1]
    grid = (B, E // be)
    return pl.pallas_call(
        _edge_mlp_body,
        grid=grid,
        in_specs=[
            pl.BlockSpec((1, be, H), lambda b, j: (b, j, 0)),
            pl.BlockSpec((be, ED), lambda b, j: (j, 0)),
            pl.BlockSpec((B, CD), lambda b, j: (0, 0)),
            pl.BlockSpec((ED, H), lambda b, j: (0, 0)),
            pl.BlockSpec((CD, H), lambda b, j: (0, 0)),
            pl.BlockSpec((1, H), lambda b, j: (0, 0)),
            pl.BlockSpec((H, H), lambda b, j: (0, 0)),
            pl.BlockSpec((1, H), lambda b, j: (0, 0)),
            pl.BlockSpec((H, H), lambda b, j: (0, 0)),
            pl.BlockSpec((1, H), lambda b, j: (0, 0)),
        ],
        out_specs=pl.BlockSpec((1, be, H), lambda b, j: (b, j, 0)),
        out_shape=jax.ShapeDtypeStruct((B, E, H), jnp.float32),
    )(g, edge_features, conditioning, w1e, w1c, b1, w2, b2, w3, b3)


def _make_scatter_kernel(B, E, NR, H):
    nch = E // (NUM_TILES * CHS)
    nzc = NR // CHS
    zc_per_tile = -(-nzc // NUM_TILES)
    mesh = plsc.VectorSubcoreMesh(core_axis_name="c", subcore_axis_name="s")

    @functools.partial(
        pl.kernel,
        mesh=mesh,
        out_type=jax.ShapeDtypeStruct((B * NR, H), jnp.float32),
        scratch_types=[
            pltpu.VMEM((nch, CHS), jnp.int32),
            pltpu.VMEM((CHS, H), jnp.float32),
            pltpu.VMEM((CHS, H), jnp.float32),
            pltpu.VMEM_SHARED((NR, H), jnp.float32),
            pltpu.SemaphoreType.DMA,
            pltpu.SemaphoreType.DMA,
            pltpu.SemaphoreType.DMA,
            pltpu.SemaphoreType.DMA,
        ],
    )
    def scatter_kernel(msg_hbm, r3d, agg_hbm, ridx, ma, mb,
                       acc, sem_la, sem_lb, sem_sa, sem_sb):
        c = lax.axis_index("c")
        s = lax.axis_index("s")
        npair = nch // 2

        def zrow(i, _):
            for k in range(H // 16):
                ma[i, pl.ds(k * 16, 16)] = jnp.zeros((16,), jnp.float32)
            return None
        lax.fori_loop(0, CHS, zrow, None)

        def zchunk(t, _):
            j = s + t * NUM_TILES

            @pl.when(j < nzc)
            def _():
                pltpu.sync_copy(ma, acc.at[pl.ds(j * CHS, CHS)])
            return None
        lax.fori_loop(0, zc_per_tile, zchunk, None)
        plsc.subcore_barrier()

        pltpu.sync_copy(r3d.at[s], ridx)
        base = c * E + s * (nch * CHS)

        def lstart(j, buf, sem):
            pltpu.async_copy(msg_hbm.at[pl.ds(base + j * CHS, CHS)], buf, sem)

        def lwait(buf, sem):
            pltpu.make_async_copy(msg_hbm.at[pl.ds(base, CHS)], buf, sem).wait()

        def sstart(j, buf, sem):
            pltpu.async_copy(buf, acc.at[ridx.at[j]], sem, add=True)

        def swait(buf, sem):
            pltpu.make_async_copy(buf, acc.at[ridx.at[0]], sem).wait()

        lstart(0, ma, sem_la)
        lstart(1, mb, sem_lb)

        def pair(p, _):
            lwait(ma, sem_la)
            sstart(2 * p, ma, sem_sa)
            lwait(mb, sem_lb)
            sstart(2 * p + 1, mb, sem_sb)
            swait(ma, sem_sa)
            pltpu.async_copy(msg_hbm.at[pl.ds(base + (2 * p + 2) * CHS, CHS)],
                             ma, sem_la)
            swait(mb, sem_sb)

            @pl.when(p < npair - 1)
            def _():
                lstart(2 * p + 3, mb, sem_lb)
            return None
        lax.fori_loop(0, npair, pair, None)

        lwait(ma, sem_la)
        sstart(nch - 1, ma, sem_sa)
        swait(ma, sem_sa)
        plsc.subcore_barrier()

        def wchunk(t, _):
            j = s + t * NUM_TILES

            @pl.when(j < nzc)
            def _():
                pltpu.sync_copy(acc.at[pl.ds(j * CHS, CHS)], ma)
                pltpu.sync_copy(ma, agg_hbm.at[pl.ds(c * NR + j * CHS, CHS)])
            return None
        lax.fori_loop(0, zc_per_tile, wchunk, None)

    return scatter_kernel


def _node_update_body(rf_ref, agg_ref, cnt_ref, cond_ref, lng_ref, lnb_ref,
                      u1_ref, c1_ref, u2_ref, c2_ref, u3_ref, c3_ref, out_ref,
                      *, RD, H, CD):
    upd_in = RD + H + CD
    rf = rf_ref[0]
    counts = jnp.maximum(cnt_ref[0][:, 0:1] + cnt_ref[1][:, 0:1], 1.0)
    a = agg_ref[0] / counts
    cond = cond_ref[pl.ds(pl.program_id(0), 1), :]

    mu = (jnp.sum(rf, axis=1, keepdims=True) + jnp.sum(a, axis=1, keepdims=True)
          + jnp.sum(cond)) / upd_in
    m2 = (jnp.sum(rf * rf, axis=1, keepdims=True)
          + jnp.sum(a * a, axis=1, keepdims=True) + jnp.sum(cond * cond)) / upd_in
    var = m2 - mu * mu
    rstd = lax.rsqrt(var + 1e-5)

    nr = (rf - mu) * rstd * lng_ref[:, 0:RD] + lnb_ref[:, 0:RD]
    na = (a - mu) * rstd * lng_ref[:, RD:RD + H] + lnb_ref[:, RD:RD + H]
    nc = (cond - mu) * rstd * lng_ref[:, RD + H:] + lnb_ref[:, RD + H:]

    h = jnp.dot(nr, u1_ref[0:RD, :], preferred_element_type=jnp.float32)
    h += jnp.dot(na, u1_ref[RD:RD + H, :], preferred_element_type=jnp.float32)
    h += jnp.dot(nc, u1_ref[RD + H:, :], preferred_element_type=jnp.float32)
    h = jnp.maximum(h + c1_ref[...], 0.0)
    h = jnp.maximum(
        jnp.dot(h, u2_ref[...], preferred_element_type=jnp.float32) + c2_ref[...], 0.0)
    out_ref[0] = rf + jnp.dot(h, u3_ref[...], preferred_element_type=jnp.float32) + c3_ref[...]


def _node_update(receiver_features, aggsum, cnt_t, conditioning, lng, lnb,
                 u1, c1, u2, c2, u3, c3, bn):
    B, NR, RD = receiver_features.shape
    H = aggsum.shape[2]
    CD = conditioning.shape[1]
    upd_in = RD + H + CD
    grid = (B, NR // bn)
    body = functools.partial(_node_update_body, RD=RD, H=H, CD=CD)
    return pl.pallas_call(
        body,
        grid=grid,
        in_specs=[
            pl.BlockSpec((1, bn, RD), lambda b, j: (b, j, 0)),
            pl.BlockSpec((1, bn, H), lambda b, j: (b, j, 0)),
            pl.BlockSpec((2, bn, H), lambda b, j: (0, j, 0)),
            pl.BlockSpec((B, CD), lambda b, j: (0, 0)),
            pl.BlockSpec((1, upd_in), lambda b, j: (0, 0)),
            pl.BlockSpec((1, upd_in), lambda b, j: (0, 0)),
            pl.BlockSpec((upd_in, H), lambda b, j: (0, 0)),
            pl.BlockSpec((1, H), lambda b, j: (0, 0)),
            pl.BlockSpec((H, H), lambda b, j: (0, 0)),
            pl.BlockSpec((1, H), lambda b, j: (0, 0)),
            pl.BlockSpec((H, RD), lambda b, j: (0, 0)),
            pl.BlockSpec((1, RD), lambda b, j: (0, 0)),
        ],
        out_specs=pl.BlockSpec((1, bn, RD), lambda b, j: (b, j, 0)),
        out_shape=jax.ShapeDtypeStruct((B, NR, RD), jnp.float32),
    )(receiver_features, aggsum, cnt_t, conditioning, lng, lnb,
      u1, c1, u2, c2, u3, c3)


def kernel(sender_features, receiver_features, edge_features, senders, receivers,
           conditioning, msg_W1, msg_b1, msg_W2, msg_b2, msg_W3, msg_b3,
           upd_W1, upd_b1, upd_W2, upd_b2, upd_W3, upd_b3, ln_gamma, ln_beta):
    B, NS, SD = sender_features.shape
    _, NR, RD = receiver_features.shape
    E, ED = edge_features.shape
    CD = conditioning.shape[1]
    H = msg_W2.shape[0]
    assert B == 2 and NS == NR and SD == RD
    assert E % (NUM_TILES * SUPG * CHG) == 0 and NR % CHG == 0
    assert E % (NUM_TILES * CHS) == 0 and NR % CHS == 0

    w1s = msg_W1[:SD]
    w1r = msg_W1[SD:SD + RD]
    w1e = msg_W1[SD + RD:SD + RD + ED]
    w1c = msg_W1[SD + RD + ED:]
    b1 = msg_b1.reshape(1, H)
    b2 = msg_b2.reshape(1, H)
    b3 = msg_b3.reshape(1, H)
    c1 = upd_b1.reshape(1, H)
    c2 = upd_b2.reshape(1, H)
    c3 = upd_b3.reshape(1, RD)
    lng = ln_gamma.reshape(1, -1)
    lnb = ln_beta.reshape(1, -1)
    nchg = E // (NUM_TILES * CHG)
    s4d = senders.astype(jnp.int32).reshape(NUM_TILES, nchg, 1, CHG)
    r4d = receivers.astype(jnp.int32).reshape(NUM_TILES, nchg, 1, CHG)
    r3d = receivers.astype(jnp.int32).reshape(NUM_TILES, E // (NUM_TILES * CHS), CHS)

    sp, rp = _node_projections(sender_features, receiver_features, w1s, w1r, bn=10000)

    gather = _make_gather_kernel(B, E, NS, NR, H)
    g, cnt = gather(sp[0], rp[0], sp[1], rp[1], s4d, r4d)
    g = g.reshape(B, E, H)
    cnt = cnt.reshape(2, NR, H)

    msg = _edge_mlp(g, edge_features, conditioning, w1e, w1c, b1,
                    msg_W2, b2, msg_W3, b3, be=16000)

    scatter = _make_scatter_kernel(B, E, NR, H)
    aggsum = scatter(msg.reshape(B * E, H), r3d).reshape(B, NR, H)

    return _node_update(receiver_features, aggsum, cnt, conditioning, lng, lnb,
                        upd_W1, c1, upd_W2, c2, upd_W3, c3, bn=5000)
